# Initial kernel scaffold; baseline (speedup 1.0000x reference)
import jax
import jax.numpy as jnp
from jax.experimental import pallas as pl


def kernel(edge_index, edge_feats, node_feats, We, be, W1, b1, W2, b2, Wih, Whh, bih, bhh):
    raise NotImplementedError("write your pallas kernel here")


# speedup vs baseline: 1.0031x; 1.0031x over previous
import functools

import jax
import jax.numpy as jnp
from jax.experimental import pallas as pl
from jax.experimental.pallas import tpu as pltpu

N = 10000
E = 320000
T = 8
D = 128
DE = 16
H = 128

BN = 2000
NB = N // BN


def _gru_body(x_ref, wih_ref, whh_ref, bih_ref, bhh_ref, out_ref, h_ref):
    t = pl.program_id(0)
    nb = pl.program_id(1)
    x = x_ref[0]
    h = jnp.where(t == 0, jnp.zeros_like(h_ref[nb]), h_ref[nb])
    gi = jnp.dot(x, wih_ref[...], preferred_element_type=jnp.float32) + bih_ref[...]
    gh = jnp.dot(h, whh_ref[...], preferred_element_type=jnp.float32) + bhh_ref[...]
    i_r, i_z, i_n = gi[:, :H], gi[:, H:2 * H], gi[:, 2 * H:]
    h_r, h_z, h_n = gh[:, :H], gh[:, H:2 * H], gh[:, 2 * H:]
    r = jax.nn.sigmoid(i_r + h_r)
    z = jax.nn.sigmoid(i_z + h_z)
    n = jnp.tanh(i_n + r * h_n)
    hn = (1.0 - z) * n + z * h
    h_ref[nb] = hn
    out_ref[0] = hn


def _gru(gnn_out, WihT, WhhT, bih, bhh):
    return pl.pallas_call(
        _gru_body,
        grid=(T, NB),
        in_specs=[
            pl.BlockSpec((1, BN, H), lambda t, nb: (t, nb, 0)),
            pl.BlockSpec((H, 3 * H), lambda t, nb: (0, 0)),
            pl.BlockSpec((H, 3 * H), lambda t, nb: (0, 0)),
            pl.BlockSpec((1, 3 * H), lambda t, nb: (0, 0)),
            pl.BlockSpec((1, 3 * H), lambda t, nb: (0, 0)),
        ],
        out_specs=pl.BlockSpec((1, BN, H), lambda t, nb: (t, nb, 0)),
        out_shape=jax.ShapeDtypeStruct((T, N, H), jnp.float32),
        scratch_shapes=[pltpu.VMEM((NB, BN, H), jnp.float32)],
    )(gnn_out, WihT, WhhT, bih, bhh)


def kernel(edge_index, edge_feats, node_feats, We, be, W1, b1, W2, b2, Wih, Whh, bih, bhh):
    w = jnp.logaddexp(edge_feats @ We + be, 0.0)[:, 0]
    row = edge_index[0]
    col = edge_index[1]
    nodes = jnp.arange(N)
    row_f = jnp.concatenate([row, nodes])
    col_f = jnp.concatenate([col, nodes])
    w_f = jnp.concatenate([w, jnp.ones((N,), w.dtype)])
    deg = jnp.zeros((N,), w.dtype).at[col_f].add(w_f)
    dinv = jnp.where(deg > 0, 1.0 / jnp.sqrt(deg), 0.0)
    norm = dinv[row_f] * w_f * dinv[col_f]

    def conv(x, W, b):
        xw = x @ W
        out = jnp.zeros((N, W.shape[1]), xw.dtype).at[col_f].add(norm[:, None] * xw[row_f])
        return out + b

    def gnn_step(x):
        h = conv(x, W1, b1)
        h = conv(jnp.tanh(h), W2, b2)
        return h + x

    gnn_out = jax.vmap(gnn_step)(node_feats)

    return _gru(gnn_out, Wih.T, Whh.T, bih[None, :], bhh[None, :])


# speedup vs baseline: 5.1821x; 5.1659x over previous
import functools

import jax
import jax.numpy as jnp
from jax import lax
from jax.experimental import pallas as pl
from jax.experimental.pallas import tpu as pltpu
from jax.experimental.pallas import tpu_sc as plsc

N = 10000
E = 320000
T = 8
D = 128
DE = 16
H = 128

NP = 10240
NC = 2
NS = 16
NW = NC * NS
C = 128
EF = E + N
EW = 10752
EP = EW * NW
CH = EW // C
RPT = NP // NS

_SC_MESH = plsc.VectorSubcoreMesh(core_axis_name="c", subcore_axis_name="s")
_SC_PARAMS = pltpu.CompilerParams(needs_layout_passes=False)

BN = 2048
NBK = NP // BN


def _deg_body(col_hbm, w_hbm, out_hbm, colbuf, wbuf, acc):
    cid = lax.axis_index("c")
    sid = lax.axis_index("s")
    wid = cid * NS + sid

    def _zero(i, _):
        acc[pl.ds(i * 16, 16)] = jnp.zeros((16,), jnp.float32)
        return 0

    lax.fori_loop(0, NP // 16, _zero, 0)

    def _chunk(g, _):
        base = wid * EW + g * C
        pltpu.sync_copy(col_hbm.at[pl.ds(base, C)], colbuf)
        pltpu.sync_copy(w_hbm.at[pl.ds(base, C)], wbuf)

        def _grp(k, _):
            idx = colbuf[pl.ds(k * 16, 16)]
            val = wbuf[pl.ds(k * 16, 16)]
            plsc.addupdate_scatter(acc, [idx], val)
            return 0

        lax.fori_loop(0, C // 16, _grp, 0)
        return 0

    lax.fori_loop(0, CH, _chunk, 0)
    pltpu.sync_copy(acc, out_hbm.at[wid])


_deg_sc = functools.partial(
    pl.kernel,
    _deg_body,
    out_type=jax.ShapeDtypeStruct((NW, NP), jnp.float32),
    mesh=_SC_MESH,
    scratch_types=[
        pltpu.VMEM((C,), jnp.int32),
        pltpu.VMEM((C,), jnp.float32),
        pltpu.VMEM((NP,), jnp.float32),
    ],
    compiler_params=_SC_PARAMS,
)()


def _norm_body(row_hbm, col_hbm, w_hbm, dinv_hbm, out_hbm,
               rowbuf, colbuf, wbuf, nbuf, dinv_v):
    cid = lax.axis_index("c")
    sid = lax.axis_index("s")
    wid = cid * NS + sid
    pltpu.sync_copy(dinv_hbm, dinv_v)

    def _chunk(g, _):
        base = wid * EW + g * C
        pltpu.sync_copy(row_hbm.at[pl.ds(base, C)], rowbuf)
        pltpu.sync_copy(col_hbm.at[pl.ds(base, C)], colbuf)
        pltpu.sync_copy(w_hbm.at[pl.ds(base, C)], wbuf)

        def _grp(k, _):
            sl = pl.ds(k * 16, 16)
            dr = plsc.load_gather(dinv_v, [rowbuf[sl]])
            dc = plsc.load_gather(dinv_v, [colbuf[sl]])
            nbuf[sl] = dr * wbuf[sl] * dc
            return 0

        lax.fori_loop(0, C // 16, _grp, 0)
        pltpu.sync_copy(nbuf, out_hbm.at[pl.ds(base, C)])
        return 0

    lax.fori_loop(0, CH, _chunk, 0)


_norm_sc = functools.partial(
    pl.kernel,
    _norm_body,
    out_type=jax.ShapeDtypeStruct((EP,), jnp.float32),
    mesh=_SC_MESH,
    scratch_types=[
        pltpu.VMEM((C,), jnp.int32),
        pltpu.VMEM((C,), jnp.int32),
        pltpu.VMEM((C,), jnp.float32),
        pltpu.VMEM((C,), jnp.float32),
        pltpu.VMEM((NP,), jnp.float32),
    ],
    compiler_params=_SC_PARAMS,
)()


def _spmm_body(xw_hbm, row_hbm, col_hbm, norm_hbm, out_hbm,
               rowbuf, colbuf, nbuf, idxbuf, gbuf, zbuf, acc, sem):
    cid = lax.axis_index("c")
    sid = lax.axis_index("s")
    wid = cid * NS + sid

    def _zrow(i, _):
        for j in range(H // 16):
            zbuf[i, pl.ds(j * 16, 16)] = jnp.zeros((16,), jnp.float32)
        return 0

    lax.fori_loop(0, C, _zrow, 0)

    def _step(t, _):
        def _zcp(i, _):
            pltpu.sync_copy(zbuf, acc.at[pl.ds(sid * RPT + i * C, C)])
            return 0

        lax.fori_loop(0, RPT // C, _zcp, 0)
        plsc.subcore_barrier()

        def _chunk(g, _):
            base = wid * EW + g * C
            pltpu.sync_copy(row_hbm.at[pl.ds(base, C)], rowbuf)
            pltpu.sync_copy(col_hbm.at[pl.ds(base, C)], colbuf)
            pltpu.sync_copy(norm_hbm.at[pl.ds(base, C)], nbuf)

            def _off(k, _):
                sl = pl.ds(k * 16, 16)
                idxbuf[sl] = rowbuf[sl] + t * NP
                return 0

            lax.fori_loop(0, C // 16, _off, 0)
            pltpu.async_copy(xw_hbm.at[idxbuf], gbuf, sem).wait()

            def _scale(e, _):
                nsplat = plsc.load_gather(nbuf, [jnp.broadcast_to(e, (16,))])
                for j in range(H // 16):
                    sl = pl.ds(j * 16, 16)
                    gbuf[e, sl] = gbuf[e, sl] * nsplat
                return 0

            lax.fori_loop(0, C, _scale, 0)
            pltpu.sync_copy(gbuf, acc.at[colbuf], add=True)
            return 0

        lax.fori_loop(0, CH, _chunk, 0)
        plsc.subcore_barrier()
        pltpu.sync_copy(acc.at[pl.ds(sid * RPT, RPT)],
                        out_hbm.at[cid, t, pl.ds(sid * RPT, RPT)])
        plsc.subcore_barrier()
        return 0

    lax.fori_loop(0, T, _step, 0)


_spmm_sc = functools.partial(
    pl.kernel,
    _spmm_body,
    out_type=jax.ShapeDtypeStruct((NC, T, NP, H), jnp.float32),
    mesh=_SC_MESH,
    scratch_types=[
        pltpu.VMEM((C,), jnp.int32),
        pltpu.VMEM((C,), jnp.int32),
        pltpu.VMEM((C,), jnp.float32),
        pltpu.VMEM((C,), jnp.int32),
        pltpu.VMEM((C, H), jnp.float32),
        pltpu.VMEM((C, H), jnp.float32),
        pltpu.VMEM_SHARED((NP, H), jnp.float32),
        pltpu.SemaphoreType.DMA,
    ],
    compiler_params=_SC_PARAMS,
)()


def _ew_body(ef_ref, web_ref, be_ref, out_ref):
    out_ref[...] = jnp.logaddexp(
        jnp.dot(ef_ref[...], web_ref[...], preferred_element_type=jnp.float32)
        + be_ref[0, 0], 0.0)


def _ew_tc(ef2, We_big, be):
    RB = 4000
    nb = ef2.shape[0] // RB
    return pl.pallas_call(
        _ew_body,
        grid=(nb,),
        in_specs=[
            pl.BlockSpec((RB, 8 * DE), lambda i: (i, 0)),
            pl.BlockSpec((8 * DE, 8), lambda i: (0, 0)),
            pl.BlockSpec((1, 1), lambda i: (0, 0), memory_space=pltpu.SMEM),
        ],
        out_specs=pl.BlockSpec((RB, 8), lambda i: (i, 0)),
        out_shape=jax.ShapeDtypeStruct((ef2.shape[0], 8), jnp.float32),
    )(ef2, We_big, be)


def _dinv_body(dp_ref, out_ref):
    deg = jnp.sum(dp_ref[...], axis=0, keepdims=True)
    out_ref[...] = jnp.where(deg > 0, jax.lax.rsqrt(deg), 0.0)


def _dinv_tc(deg_part):
    return pl.pallas_call(
        _dinv_body,
        grid=(NBK,),
        in_specs=[pl.BlockSpec((NW, BN), lambda i: (0, i))],
        out_specs=pl.BlockSpec((1, BN), lambda i: (0, i)),
        out_shape=jax.ShapeDtypeStruct((1, NP), jnp.float32),
    )(deg_part)


def _mm_body(x_ref, w_ref, out_ref):
    out_ref[0] = jnp.dot(x_ref[0], w_ref[...], preferred_element_type=jnp.float32)


def _mm_tc(xp, W):
    return pl.pallas_call(
        _mm_body,
        grid=(T, NBK),
        in_specs=[
            pl.BlockSpec((1, BN, H), lambda t, nb: (t, nb, 0)),
            pl.BlockSpec((H, H), lambda t, nb: (0, 0)),
        ],
        out_specs=pl.BlockSpec((1, BN, H), lambda t, nb: (t, nb, 0)),
        out_shape=jax.ShapeDtypeStruct((T, NP, H), jnp.float32),
    )(xp, W)


def _cmb_body(p_ref, b_ref, w_ref, out_ref):
    h1 = jnp.tanh(p_ref[0, 0] + p_ref[1, 0] + b_ref[...])
    out_ref[0] = jnp.dot(h1, w_ref[...], preferred_element_type=jnp.float32)


def _cmb_tc(parts, b1, W2):
    return pl.pallas_call(
        _cmb_body,
        grid=(T, NBK),
        in_specs=[
            pl.BlockSpec((NC, 1, BN, H), lambda t, nb: (0, t, nb, 0)),
            pl.BlockSpec((1, H), lambda t, nb: (0, 0)),
            pl.BlockSpec((H, H), lambda t, nb: (0, 0)),
        ],
        out_specs=pl.BlockSpec((1, BN, H), lambda t, nb: (t, nb, 0)),
        out_shape=jax.ShapeDtypeStruct((T, NP, H), jnp.float32),
    )(parts, b1, W2)


def _gru_body(p_ref, x_ref, b_ref, wih_ref, whh_ref, bih_ref, bhh_ref,
              out_ref, h_ref):
    t = pl.program_id(0)
    nb = pl.program_id(1)
    g = p_ref[0, 0] + p_ref[1, 0] + b_ref[...] + x_ref[0]
    h = jnp.where(t == 0, jnp.zeros_like(h_ref[nb]), h_ref[nb])
    gi = jnp.dot(g, wih_ref[...], preferred_element_type=jnp.float32) + bih_ref[...]
    gh = jnp.dot(h, whh_ref[...], preferred_element_type=jnp.float32) + bhh_ref[...]
    i_r, i_z, i_n = gi[:, :H], gi[:, H:2 * H], gi[:, 2 * H:]
    h_r, h_z, h_n = gh[:, :H], gh[:, H:2 * H], gh[:, 2 * H:]
    r = jax.nn.sigmoid(i_r + h_r)
    z = jax.nn.sigmoid(i_z + h_z)
    n = jnp.tanh(i_n + r * h_n)
    hn = (1.0 - z) * n + z * h
    h_ref[nb] = hn
    out_ref[0] = hn


def _gru_tc(parts, xp, b2, WihT, WhhT, bih, bhh):
    return pl.pallas_call(
        _gru_body,
        grid=(T, NBK),
        in_specs=[
            pl.BlockSpec((NC, 1, BN, H), lambda t, nb: (0, t, nb, 0)),
            pl.BlockSpec((1, BN, H), lambda t, nb: (t, nb, 0)),
            pl.BlockSpec((1, H), lambda t, nb: (0, 0)),
            pl.BlockSpec((H, 3 * H), lambda t, nb: (0, 0)),
            pl.BlockSpec((H, 3 * H), lambda t, nb: (0, 0)),
            pl.BlockSpec((1, 3 * H), lambda t, nb: (0, 0)),
            pl.BlockSpec((1, 3 * H), lambda t, nb: (0, 0)),
        ],
        out_specs=pl.BlockSpec((1, BN, H), lambda t, nb: (t, nb, 0)),
        out_shape=jax.ShapeDtypeStruct((T, NP, H), jnp.float32),
        scratch_shapes=[pltpu.VMEM((NBK, BN, H), jnp.float32)],
    )(parts, xp, b2, WihT, WhhT, bih, bhh)


def kernel(edge_index, edge_feats, node_feats, We, be, W1, b1, W2, b2, Wih, Whh, bih, bhh):
    row = edge_index[0]
    col = edge_index[1]

    We_big = jnp.kron(jnp.eye(8, dtype=We.dtype), We)
    ef2 = edge_feats.reshape(E // 8, 8 * DE)
    w = _ew_tc(ef2, We_big, be.reshape(1, 1)).reshape(E)

    nodes = jnp.arange(N, dtype=row.dtype)
    pad = EP - EF
    row_f = jnp.concatenate([row, nodes, jnp.zeros((pad,), row.dtype)])
    col_f = jnp.concatenate([col, nodes, jnp.zeros((pad,), col.dtype)])
    w_f = jnp.concatenate([w, jnp.ones((N,), w.dtype), jnp.zeros((pad,), w.dtype)])

    deg_part = _deg_sc(col_f, w_f)
    dinv = _dinv_tc(deg_part).reshape(NP)
    norm = _norm_sc(row_f, col_f, w_f, dinv)

    xp = jnp.pad(node_feats, ((0, 0), (0, NP - N), (0, 0)))

    xw1 = _mm_tc(xp, W1)
    p1 = _spmm_sc(xw1.reshape(T * NP, H), row_f, col_f, norm)
    xw2 = _cmb_tc(p1, b1.reshape(1, H), W2)
    p2 = _spmm_sc(xw2.reshape(T * NP, H), row_f, col_f, norm)

    seq = _gru_tc(p2, xp, b2.reshape(1, H), Wih.T, Whh.T,
                  bih.reshape(1, 3 * H), bhh.reshape(1, 3 * H))
    return seq[:, :N, :]


# speedup vs baseline: 15.4421x; 2.9799x over previous
import functools

import jax
import jax.numpy as jnp
from jax import lax
from jax.experimental import pallas as pl
from jax.experimental.pallas import tpu as pltpu
from jax.experimental.pallas import tpu_sc as plsc

N = 10000
E = 320000
T = 8
D = 128
DE = 16
H = 128

NP = 10240
NC = 2
NS = 16
NW = NC * NS
C = 128
EF = E + N
EW = 10752
EP = EW * NW
CH = EW // C
RPT = NP // NS

_SC_MESH = plsc.VectorSubcoreMesh(core_axis_name="c", subcore_axis_name="s")
_SC_PARAMS = pltpu.CompilerParams(needs_layout_passes=False)

BN = 2048
NBK = NP // BN


def _deg_body(col_hbm, w_hbm, out_hbm, colbuf, wbuf, acc):
    cid = lax.axis_index("c")
    sid = lax.axis_index("s")
    wid = cid * NS + sid

    def _zero(i, _):
        acc[pl.ds(i * 16, 16)] = jnp.zeros((16,), jnp.float32)
        return 0

    lax.fori_loop(0, NP // 16, _zero, 0)

    def _chunk(g, _):
        base = wid * EW + g * C
        pltpu.sync_copy(col_hbm.at[pl.ds(base, C)], colbuf)
        pltpu.sync_copy(w_hbm.at[pl.ds(base, C)], wbuf)

        def _grp(k, _):
            idx = colbuf[pl.ds(k * 16, 16)]
            val = wbuf[pl.ds(k * 16, 16)]
            plsc.addupdate_scatter(acc, [idx], val)
            return 0

        lax.fori_loop(0, C // 16, _grp, 0)
        return 0

    lax.fori_loop(0, CH, _chunk, 0)
    pltpu.sync_copy(acc, out_hbm.at[wid])


_deg_sc = functools.partial(
    pl.kernel,
    _deg_body,
    out_type=jax.ShapeDtypeStruct((NW, NP), jnp.float32),
    mesh=_SC_MESH,
    scratch_types=[
        pltpu.VMEM((C,), jnp.int32),
        pltpu.VMEM((C,), jnp.float32),
        pltpu.VMEM((NP,), jnp.float32),
    ],
    compiler_params=_SC_PARAMS,
)()


def _norm_body(row_hbm, col_hbm, w_hbm, dinv_hbm, out_hbm,
               rowbuf, colbuf, wbuf, nbuf, dinv_v):
    cid = lax.axis_index("c")
    sid = lax.axis_index("s")
    wid = cid * NS + sid
    pltpu.sync_copy(dinv_hbm, dinv_v)

    def _chunk(g, _):
        base = wid * EW + g * C
        pltpu.sync_copy(row_hbm.at[pl.ds(base, C)], rowbuf)
        pltpu.sync_copy(col_hbm.at[pl.ds(base, C)], colbuf)
        pltpu.sync_copy(w_hbm.at[pl.ds(base, C)], wbuf)

        def _grp(k, _):
            sl = pl.ds(k * 16, 16)
            dr = plsc.load_gather(dinv_v, [rowbuf[sl]])
            dc = plsc.load_gather(dinv_v, [colbuf[sl]])
            nbuf[sl] = dr * wbuf[sl] * dc
            return 0

        lax.fori_loop(0, C // 16, _grp, 0)
        pltpu.sync_copy(nbuf, out_hbm.at[pl.ds(base, C)])
        return 0

    lax.fori_loop(0, CH, _chunk, 0)


_norm_sc = functools.partial(
    pl.kernel,
    _norm_body,
    out_type=jax.ShapeDtypeStruct((EP,), jnp.float32),
    mesh=_SC_MESH,
    scratch_types=[
        pltpu.VMEM((C,), jnp.int32),
        pltpu.VMEM((C,), jnp.int32),
        pltpu.VMEM((C,), jnp.float32),
        pltpu.VMEM((C,), jnp.float32),
        pltpu.VMEM((NP,), jnp.float32),
    ],
    compiler_params=_SC_PARAMS,
)()


def _spmm_body(xw_hbm, row_hbm, col_hbm, norm_hbm, out_hbm,
               idx_a, idx_b, col_a, col_b, nrm_a, nrm_b, gbuf_a, gbuf_b,
               acc, gsem_a, gsem_b, ssem_a, ssem_b):
    cid = lax.axis_index("c")
    sid = lax.axis_index("s")
    wid = cid * NS + sid
    wbase = wid * EW

    idx = (idx_a, idx_b)
    cols = (col_a, col_b)
    nrms = (nrm_a, nrm_b)
    gbufs = (gbuf_a, gbuf_b)
    gsems = (gsem_a, gsem_b)
    ssems = (ssem_a, ssem_b)

    def _load_issue(t, g, s):
        base = wbase + g * C
        pltpu.sync_copy(row_hbm.at[pl.ds(base, C)], idx[s])
        pltpu.sync_copy(col_hbm.at[pl.ds(base, C)], cols[s])
        pltpu.sync_copy(norm_hbm.at[pl.ds(base, C)], nrms[s])

        def _off(k, _):
            sl = pl.ds(k * 16, 16)
            idx[s][sl] = idx[s][sl] + t * NP
            return 0

        lax.fori_loop(0, C // 16, _off, 0)
        pltpu.async_copy(xw_hbm.at[idx[s]], gbufs[s], gsems[s])

    def _scale(s):
        def _grp(k, _):
            nv = nrms[s][pl.ds(k * 16, 16)]
            for l in range(16):
                nsplat = jnp.broadcast_to(nv[l], (16,))
                for j in range(H // 16):
                    sl = pl.ds(j * 16, 16)
                    gbufs[s][k * 16 + l, sl] = gbufs[s][k * 16 + l, sl] * nsplat
            return 0

        lax.fori_loop(0, C // 16, _grp, 0)

    def _wait_gather(s):
        pltpu.make_async_copy(xw_hbm.at[idx[s]], gbufs[s], gsems[s]).wait()

    def _issue_scatter(s):
        pltpu.async_copy(gbufs[s], acc.at[cols[s]], ssems[s], add=True)

    def _wait_scatter(s):
        pltpu.make_async_copy(gbufs[s], acc.at[cols[s]], ssems[s]).wait()

    def _step(t, _):
        def _zrow(i, _):
            for j in range(H // 16):
                gbuf_a[i, pl.ds(j * 16, 16)] = jnp.zeros((16,), jnp.float32)
            return 0

        lax.fori_loop(0, C, _zrow, 0)

        def _zcp(i, _):
            pltpu.sync_copy(gbuf_a, acc.at[pl.ds(sid * RPT + i * C, C)])
            return 0

        lax.fori_loop(0, RPT // C, _zcp, 0)
        plsc.subcore_barrier()

        _load_issue(t, 0, 0)

        def _pair(gg, _):
            e = 2 * gg
            _wait_gather(0)
            _scale(0)
            _issue_scatter(0)

            @pl.when(gg >= 1)
            def _():
                _wait_scatter(1)

            _load_issue(t, e + 1, 1)
            _wait_gather(1)
            _scale(1)
            _issue_scatter(1)
            _wait_scatter(0)

            @pl.when(gg < CH // 2 - 1)
            def _():
                _load_issue(t, e + 2, 0)

            return 0

        lax.fori_loop(0, CH // 2, _pair, 0)
        _wait_scatter(1)
        plsc.subcore_barrier()
        pltpu.sync_copy(acc.at[pl.ds(sid * RPT, RPT)],
                        out_hbm.at[cid, t, pl.ds(sid * RPT, RPT)])
        plsc.subcore_barrier()
        return 0

    lax.fori_loop(0, T, _step, 0)


_spmm_sc = functools.partial(
    pl.kernel,
    _spmm_body,
    out_type=jax.ShapeDtypeStruct((NC, T, NP, H), jnp.float32),
    mesh=_SC_MESH,
    scratch_types=[
        pltpu.VMEM((C,), jnp.int32),
        pltpu.VMEM((C,), jnp.int32),
        pltpu.VMEM((C,), jnp.int32),
        pltpu.VMEM((C,), jnp.int32),
        pltpu.VMEM((C,), jnp.float32),
        pltpu.VMEM((C,), jnp.float32),
        pltpu.VMEM((C, H), jnp.float32),
        pltpu.VMEM((C, H), jnp.float32),
        pltpu.VMEM_SHARED((NP, H), jnp.float32),
        pltpu.SemaphoreType.DMA,
        pltpu.SemaphoreType.DMA,
        pltpu.SemaphoreType.DMA,
        pltpu.SemaphoreType.DMA,
    ],
    compiler_params=_SC_PARAMS,
)()


def _ew_body(ef_ref, web_ref, be_ref, out_ref):
    out_ref[...] = jnp.logaddexp(
        jnp.dot(ef_ref[...], web_ref[...], preferred_element_type=jnp.float32)
        + be_ref[0, 0], 0.0)


def _ew_tc(ef2, We_big, be):
    RB = 4000
    nb = ef2.shape[0] // RB
    return pl.pallas_call(
        _ew_body,
        grid=(nb,),
        in_specs=[
            pl.BlockSpec((RB, 8 * DE), lambda i: (i, 0)),
            pl.BlockSpec((8 * DE, 8), lambda i: (0, 0)),
            pl.BlockSpec((1, 1), lambda i: (0, 0), memory_space=pltpu.SMEM),
        ],
        out_specs=pl.BlockSpec((RB, 8), lambda i: (i, 0)),
        out_shape=jax.ShapeDtypeStruct((ef2.shape[0], 8), jnp.float32),
    )(ef2, We_big, be)


def _dinv_body(dp_ref, out_ref):
    deg = jnp.sum(dp_ref[...], axis=0, keepdims=True)
    out_ref[...] = jnp.where(deg > 0, jax.lax.rsqrt(deg), 0.0)


def _dinv_tc(deg_part):
    return pl.pallas_call(
        _dinv_body,
        grid=(NBK,),
        in_specs=[pl.BlockSpec((NW, BN), lambda i: (0, i))],
        out_specs=pl.BlockSpec((1, BN), lambda i: (0, i)),
        out_shape=jax.ShapeDtypeStruct((1, NP), jnp.float32),
    )(deg_part)


def _mm_body(x_ref, w_ref, out_ref):
    out_ref[0] = jnp.dot(x_ref[0], w_ref[...], preferred_element_type=jnp.float32)


def _mm_tc(xp, W):
    return pl.pallas_call(
        _mm_body,
        grid=(T, NBK),
        in_specs=[
            pl.BlockSpec((1, BN, H), lambda t, nb: (t, nb, 0)),
            pl.BlockSpec((H, H), lambda t, nb: (0, 0)),
        ],
        out_specs=pl.BlockSpec((1, BN, H), lambda t, nb: (t, nb, 0)),
        out_shape=jax.ShapeDtypeStruct((T, NP, H), jnp.float32),
    )(xp, W)


def _cmb_body(p_ref, b_ref, w_ref, out_ref):
    h1 = jnp.tanh(p_ref[0, 0] + p_ref[1, 0] + b_ref[...])
    out_ref[0] = jnp.dot(h1, w_ref[...], preferred_element_type=jnp.float32)


def _cmb_tc(parts, b1, W2):
    return pl.pallas_call(
        _cmb_body,
        grid=(T, NBK),
        in_specs=[
            pl.BlockSpec((NC, 1, BN, H), lambda t, nb: (0, t, nb, 0)),
            pl.BlockSpec((1, H), lambda t, nb: (0, 0)),
            pl.BlockSpec((H, H), lambda t, nb: (0, 0)),
        ],
        out_specs=pl.BlockSpec((1, BN, H), lambda t, nb: (t, nb, 0)),
        out_shape=jax.ShapeDtypeStruct((T, NP, H), jnp.float32),
    )(parts, b1, W2)


def _gru_body(p_ref, x_ref, b_ref, wih_ref, whh_ref, bih_ref, bhh_ref,
              out_ref, h_ref):
    t = pl.program_id(0)
    nb = pl.program_id(1)
    g = p_ref[0, 0] + p_ref[1, 0] + b_ref[...] + x_ref[0]
    h = jnp.where(t == 0, jnp.zeros_like(h_ref[nb]), h_ref[nb])
    gi = jnp.dot(g, wih_ref[...], preferred_element_type=jnp.float32) + bih_ref[...]
    gh = jnp.dot(h, whh_ref[...], preferred_element_type=jnp.float32) + bhh_ref[...]
    i_r, i_z, i_n = gi[:, :H], gi[:, H:2 * H], gi[:, 2 * H:]
    h_r, h_z, h_n = gh[:, :H], gh[:, H:2 * H], gh[:, 2 * H:]
    r = jax.nn.sigmoid(i_r + h_r)
    z = jax.nn.sigmoid(i_z + h_z)
    n = jnp.tanh(i_n + r * h_n)
    hn = (1.0 - z) * n + z * h
    h_ref[nb] = hn
    out_ref[0] = hn


def _gru_tc(parts, xp, b2, WihT, WhhT, bih, bhh):
    return pl.pallas_call(
        _gru_body,
        grid=(T, NBK),
        in_specs=[
            pl.BlockSpec((NC, 1, BN, H), lambda t, nb: (0, t, nb, 0)),
            pl.BlockSpec((1, BN, H), lambda t, nb: (t, nb, 0)),
            pl.BlockSpec((1, H), lambda t, nb: (0, 0)),
            pl.BlockSpec((H, 3 * H), lambda t, nb: (0, 0)),
            pl.BlockSpec((H, 3 * H), lambda t, nb: (0, 0)),
            pl.BlockSpec((1, 3 * H), lambda t, nb: (0, 0)),
            pl.BlockSpec((1, 3 * H), lambda t, nb: (0, 0)),
        ],
        out_specs=pl.BlockSpec((1, BN, H), lambda t, nb: (t, nb, 0)),
        out_shape=jax.ShapeDtypeStruct((T, NP, H), jnp.float32),
        scratch_shapes=[pltpu.VMEM((NBK, BN, H), jnp.float32)],
    )(parts, xp, b2, WihT, WhhT, bih, bhh)


def kernel(edge_index, edge_feats, node_feats, We, be, W1, b1, W2, b2, Wih, Whh, bih, bhh):
    row = edge_index[0]
    col = edge_index[1]

    We_big = jnp.kron(jnp.eye(8, dtype=We.dtype), We)
    ef2 = edge_feats.reshape(E // 8, 8 * DE)
    w = _ew_tc(ef2, We_big, be.reshape(1, 1)).reshape(E)

    nodes = jnp.arange(N, dtype=row.dtype)
    pad = EP - EF
    spread = jnp.arange(pad, dtype=row.dtype) % N
    row_f = jnp.concatenate([row, nodes, spread])
    col_f = jnp.concatenate([col, nodes, spread])
    w_f = jnp.concatenate([w, jnp.ones((N,), w.dtype), jnp.zeros((pad,), w.dtype)])

    deg_part = _deg_sc(col_f, w_f)
    dinv = _dinv_tc(deg_part).reshape(NP)
    norm = _norm_sc(row_f, col_f, w_f, dinv)

    xp = jnp.pad(node_feats, ((0, 0), (0, NP - N), (0, 0)))

    xw1 = _mm_tc(xp, W1)
    p1 = _spmm_sc(xw1.reshape(T * NP, H), row_f, col_f, norm)
    xw2 = _cmb_tc(p1, b1.reshape(1, H), W2)
    p2 = _spmm_sc(xw2.reshape(T * NP, H), row_f, col_f, norm)

    seq = _gru_tc(p2, xp, b2.reshape(1, H), Wih.T, Whh.T,
                  bih.reshape(1, 3 * H), bhh.reshape(1, 3 * H))
    return seq[:, :N, :]


# speedup vs baseline: 20.3127x; 1.3154x over previous
import functools

import jax
import jax.numpy as jnp
from jax import lax
from jax.experimental import pallas as pl
from jax.experimental.pallas import tpu as pltpu
from jax.experimental.pallas import tpu_sc as plsc

N = 10000
E = 320000
T = 8
D = 128
DE = 16
H = 128

NP = 10240
NC = 2
NS = 16
NW = NC * NS
C = 128
EF = E + N
EW = 11264
EP = EW * NW
CH = EW // C
RPT = NP // NS

_SC_MESH = plsc.VectorSubcoreMesh(core_axis_name="c", subcore_axis_name="s")
_SC_PARAMS = pltpu.CompilerParams(needs_layout_passes=False)

BN = 2048
NBK = NP // BN


def _deg_body(col_hbm, w_hbm, out_hbm, colbuf, wbuf, acc):
    cid = lax.axis_index("c")
    sid = lax.axis_index("s")
    wid = cid * NS + sid

    def _zero(i, _):
        acc[pl.ds(i * 16, 16)] = jnp.zeros((16,), jnp.float32)
        return 0

    lax.fori_loop(0, NP // 16, _zero, 0)

    def _chunk(g, _):
        base = wid * EW + g * C
        pltpu.sync_copy(col_hbm.at[pl.ds(base, C)], colbuf)
        pltpu.sync_copy(w_hbm.at[pl.ds(base, C)], wbuf)

        def _grp(k, _):
            idx = colbuf[pl.ds(k * 16, 16)]
            val = wbuf[pl.ds(k * 16, 16)]
            plsc.addupdate_scatter(acc, [idx], val)
            return 0

        lax.fori_loop(0, C // 16, _grp, 0)
        return 0

    lax.fori_loop(0, CH, _chunk, 0)
    pltpu.sync_copy(acc, out_hbm.at[wid])


_deg_sc = functools.partial(
    pl.kernel,
    _deg_body,
    out_type=jax.ShapeDtypeStruct((NW, NP), jnp.float32),
    mesh=_SC_MESH,
    scratch_types=[
        pltpu.VMEM((C,), jnp.int32),
        pltpu.VMEM((C,), jnp.float32),
        pltpu.VMEM((NP,), jnp.float32),
    ],
    compiler_params=_SC_PARAMS,
)()


def _norm_body(row_hbm, col_hbm, w_hbm, dinv_hbm, out_hbm,
               rowbuf, colbuf, wbuf, nbuf, dinv_v):
    cid = lax.axis_index("c")
    sid = lax.axis_index("s")
    wid = cid * NS + sid
    pltpu.sync_copy(dinv_hbm, dinv_v)

    def _chunk(g, _):
        base = wid * EW + g * C
        pltpu.sync_copy(row_hbm.at[pl.ds(base, C)], rowbuf)
        pltpu.sync_copy(col_hbm.at[pl.ds(base, C)], colbuf)
        pltpu.sync_copy(w_hbm.at[pl.ds(base, C)], wbuf)

        def _grp(k, _):
            sl = pl.ds(k * 16, 16)
            dr = plsc.load_gather(dinv_v, [rowbuf[sl]])
            dc = plsc.load_gather(dinv_v, [colbuf[sl]])
            nbuf[sl] = dr * wbuf[sl] * dc
            return 0

        lax.fori_loop(0, C // 16, _grp, 0)
        pltpu.sync_copy(nbuf, out_hbm.at[pl.ds(base, C)])
        return 0

    lax.fori_loop(0, CH, _chunk, 0)


_norm_sc = functools.partial(
    pl.kernel,
    _norm_body,
    out_type=jax.ShapeDtypeStruct((EP,), jnp.float32),
    mesh=_SC_MESH,
    scratch_types=[
        pltpu.VMEM((C,), jnp.int32),
        pltpu.VMEM((C,), jnp.int32),
        pltpu.VMEM((C,), jnp.float32),
        pltpu.VMEM((C,), jnp.float32),
        pltpu.VMEM((NP,), jnp.float32),
    ],
    compiler_params=_SC_PARAMS,
)()


BLK = 8
NBL = CH // BLK


def _spmm_body(xw_hbm, row_hbm, col_hbm, norm_hbm, out_hbm,
               rowblk, colblk, nrmblk, idx_a, idx_b, gbuf_a, gbuf_b,
               acc, gsem_a, gsem_b, ssem_a, ssem_b):
    cid = lax.axis_index("c")
    sid = lax.axis_index("s")
    wid = cid * NS + sid
    wrow = wid * CH

    idx = (idx_a, idx_b)
    gbufs = (gbuf_a, gbuf_b)
    gsems = (gsem_a, gsem_b)
    ssems = (ssem_a, ssem_b)

    def _load_issue(t, lc, s):
        def _off(k, _):
            sl = pl.ds(k * 16, 16)
            idx[s][sl] = rowblk[lc, sl] + t * NP
            return 0

        lax.fori_loop(0, C // 16, _off, 0)
        pltpu.async_copy(xw_hbm.at[idx[s]], gbufs[s], gsems[s])

    def _scale(lc, s):
        def _grp(k, _):
            nv = nrmblk[lc, pl.ds(k * 16, 16)]
            for l in range(16):
                nsplat = jnp.broadcast_to(nv[l], (16,))
                for j in range(H // 16):
                    sl = pl.ds(j * 16, 16)
                    gbufs[s][k * 16 + l, sl] = gbufs[s][k * 16 + l, sl] * nsplat
            return 0

        lax.fori_loop(0, C // 16, _grp, 0)

    def _wait_gather(s):
        pltpu.make_async_copy(xw_hbm.at[idx[s]], gbufs[s], gsems[s]).wait()

    def _issue_scatter(lc, s):
        pltpu.async_copy(gbufs[s], acc.at[colblk.at[lc]], ssems[s], add=True)

    def _wait_scatter(lc, s):
        pltpu.make_async_copy(gbufs[s], acc.at[colblk.at[lc]], ssems[s]).wait()

    def _step(t, _):
        def _zrow(i, _):
            for j in range(H // 16):
                gbuf_a[i, pl.ds(j * 16, 16)] = jnp.zeros((16,), jnp.float32)
            return 0

        lax.fori_loop(0, C, _zrow, 0)

        def _zcp(i, _):
            pltpu.sync_copy(gbuf_a, acc.at[pl.ds(sid * RPT + i * C, C)])
            return 0

        lax.fori_loop(0, RPT // C, _zcp, 0)
        plsc.subcore_barrier()

        def _blk(b, _):
            brow = wrow + b * BLK
            pltpu.sync_copy(row_hbm.at[pl.ds(brow, BLK)], rowblk)
            pltpu.sync_copy(col_hbm.at[pl.ds(brow, BLK)], colblk)
            pltpu.sync_copy(norm_hbm.at[pl.ds(brow, BLK)], nrmblk)
            _load_issue(t, 0, 0)

            def _pair(pp, _):
                e = 2 * pp
                _wait_gather(0)
                _scale(e, 0)
                _issue_scatter(e, 0)

                @pl.when(pp >= 1)
                def _():
                    _wait_scatter(e - 1, 1)

                _load_issue(t, e + 1, 1)
                _wait_gather(1)
                _scale(e + 1, 1)
                _issue_scatter(e + 1, 1)
                _wait_scatter(e, 0)

                @pl.when(pp < BLK // 2 - 1)
                def _():
                    _load_issue(t, e + 2, 0)

                return 0

            lax.fori_loop(0, BLK // 2, _pair, 0)
            _wait_scatter(BLK - 1, 1)
            return 0

        lax.fori_loop(0, NBL, _blk, 0)
        plsc.subcore_barrier()
        pltpu.sync_copy(acc.at[pl.ds(sid * RPT, RPT)],
                        out_hbm.at[cid, t, pl.ds(sid * RPT, RPT)])
        plsc.subcore_barrier()
        return 0

    lax.fori_loop(0, T, _step, 0)


_spmm_sc = functools.partial(
    pl.kernel,
    _spmm_body,
    out_type=jax.ShapeDtypeStruct((NC, T, NP, H), jnp.float32),
    mesh=_SC_MESH,
    scratch_types=[
        pltpu.VMEM((BLK, C), jnp.int32),
        pltpu.VMEM((BLK, C), jnp.int32),
        pltpu.VMEM((BLK, C), jnp.float32),
        pltpu.VMEM((C,), jnp.int32),
        pltpu.VMEM((C,), jnp.int32),
        pltpu.VMEM((C, H), jnp.float32),
        pltpu.VMEM((C, H), jnp.float32),
        pltpu.VMEM_SHARED((NP, H), jnp.float32),
        pltpu.SemaphoreType.DMA,
        pltpu.SemaphoreType.DMA,
        pltpu.SemaphoreType.DMA,
        pltpu.SemaphoreType.DMA,
    ],
    compiler_params=_SC_PARAMS,
)()


def _ew_body(ef_ref, web_ref, be_ref, out_ref):
    out_ref[...] = jnp.logaddexp(
        jnp.dot(ef_ref[...], web_ref[...], preferred_element_type=jnp.float32)
        + be_ref[0, 0], 0.0)


def _ew_tc(ef2, We_big, be):
    RB = 4000
    nb = ef2.shape[0] // RB
    return pl.pallas_call(
        _ew_body,
        grid=(nb,),
        in_specs=[
            pl.BlockSpec((RB, 8 * DE), lambda i: (i, 0)),
            pl.BlockSpec((8 * DE, 8), lambda i: (0, 0)),
            pl.BlockSpec((1, 1), lambda i: (0, 0), memory_space=pltpu.SMEM),
        ],
        out_specs=pl.BlockSpec((RB, 8), lambda i: (i, 0)),
        out_shape=jax.ShapeDtypeStruct((ef2.shape[0], 8), jnp.float32),
    )(ef2, We_big, be)


def _dinv_body(dp_ref, out_ref):
    deg = jnp.sum(dp_ref[...], axis=0, keepdims=True)
    out_ref[...] = jnp.where(deg > 0, jax.lax.rsqrt(deg), 0.0)


def _dinv_tc(deg_part):
    return pl.pallas_call(
        _dinv_body,
        grid=(NBK,),
        in_specs=[pl.BlockSpec((NW, BN), lambda i: (0, i))],
        out_specs=pl.BlockSpec((1, BN), lambda i: (0, i)),
        out_shape=jax.ShapeDtypeStruct((1, NP), jnp.float32),
    )(deg_part)


def _mm_body(x_ref, w_ref, out_ref):
    out_ref[0] = jnp.dot(x_ref[0], w_ref[...], preferred_element_type=jnp.float32)


def _mm_tc(xp, W):
    return pl.pallas_call(
        _mm_body,
        grid=(T, NBK),
        in_specs=[
            pl.BlockSpec((1, BN, H), lambda t, nb: (t, nb, 0)),
            pl.BlockSpec((H, H), lambda t, nb: (0, 0)),
        ],
        out_specs=pl.BlockSpec((1, BN, H), lambda t, nb: (t, nb, 0)),
        out_shape=jax.ShapeDtypeStruct((T, NP, H), jnp.float32),
    )(xp, W)


def _cmb_body(p_ref, b_ref, w_ref, out_ref):
    h1 = jnp.tanh(p_ref[0, 0] + p_ref[1, 0] + b_ref[...])
    out_ref[0] = jnp.dot(h1, w_ref[...], preferred_element_type=jnp.float32)


def _cmb_tc(parts, b1, W2):
    return pl.pallas_call(
        _cmb_body,
        grid=(T, NBK),
        in_specs=[
            pl.BlockSpec((NC, 1, BN, H), lambda t, nb: (0, t, nb, 0)),
            pl.BlockSpec((1, H), lambda t, nb: (0, 0)),
            pl.BlockSpec((H, H), lambda t, nb: (0, 0)),
        ],
        out_specs=pl.BlockSpec((1, BN, H), lambda t, nb: (t, nb, 0)),
        out_shape=jax.ShapeDtypeStruct((T, NP, H), jnp.float32),
    )(parts, b1, W2)


def _gru_body(p_ref, x_ref, b_ref, wih_ref, whh_ref, bih_ref, bhh_ref,
              out_ref, h_ref):
    t = pl.program_id(0)
    nb = pl.program_id(1)
    g = p_ref[0, 0] + p_ref[1, 0] + b_ref[...] + x_ref[0]
    h = jnp.where(t == 0, jnp.zeros_like(h_ref[nb]), h_ref[nb])
    gi = jnp.dot(g, wih_ref[...], preferred_element_type=jnp.float32) + bih_ref[...]
    gh = jnp.dot(h, whh_ref[...], preferred_element_type=jnp.float32) + bhh_ref[...]
    i_r, i_z, i_n = gi[:, :H], gi[:, H:2 * H], gi[:, 2 * H:]
    h_r, h_z, h_n = gh[:, :H], gh[:, H:2 * H], gh[:, 2 * H:]
    r = jax.nn.sigmoid(i_r + h_r)
    z = jax.nn.sigmoid(i_z + h_z)
    n = jnp.tanh(i_n + r * h_n)
    hn = (1.0 - z) * n + z * h
    h_ref[nb] = hn
    out_ref[0] = hn


def _gru_tc(parts, xp, b2, WihT, WhhT, bih, bhh):
    return pl.pallas_call(
        _gru_body,
        grid=(T, NBK),
        in_specs=[
            pl.BlockSpec((NC, 1, BN, H), lambda t, nb: (0, t, nb, 0)),
            pl.BlockSpec((1, BN, H), lambda t, nb: (t, nb, 0)),
            pl.BlockSpec((1, H), lambda t, nb: (0, 0)),
            pl.BlockSpec((H, 3 * H), lambda t, nb: (0, 0)),
            pl.BlockSpec((H, 3 * H), lambda t, nb: (0, 0)),
            pl.BlockSpec((1, 3 * H), lambda t, nb: (0, 0)),
            pl.BlockSpec((1, 3 * H), lambda t, nb: (0, 0)),
        ],
        out_specs=pl.BlockSpec((1, BN, H), lambda t, nb: (t, nb, 0)),
        out_shape=jax.ShapeDtypeStruct((T, NP, H), jnp.float32),
        scratch_shapes=[pltpu.VMEM((NBK, BN, H), jnp.float32)],
    )(parts, xp, b2, WihT, WhhT, bih, bhh)


def kernel(edge_index, edge_feats, node_feats, We, be, W1, b1, W2, b2, Wih, Whh, bih, bhh):
    row = edge_index[0]
    col = edge_index[1]

    We_big = jnp.kron(jnp.eye(8, dtype=We.dtype), We)
    ef2 = edge_feats.reshape(E // 8, 8 * DE)
    w = _ew_tc(ef2, We_big, be.reshape(1, 1)).reshape(E)

    nodes = jnp.arange(N, dtype=row.dtype)
    pad = EP - EF
    spread = jnp.arange(pad, dtype=row.dtype) % N
    row_f = jnp.concatenate([row, nodes, spread])
    col_f = jnp.concatenate([col, nodes, spread])
    w_f = jnp.concatenate([w, jnp.ones((N,), w.dtype), jnp.zeros((pad,), w.dtype)])

    deg_part = _deg_sc(col_f, w_f)
    dinv = _dinv_tc(deg_part).reshape(NP)
    norm = _norm_sc(row_f, col_f, w_f, dinv)

    xp = jnp.pad(node_feats, ((0, 0), (0, NP - N), (0, 0)))

    row2 = row_f.reshape(EP // C, C)
    col2 = col_f.reshape(EP // C, C)
    nrm2 = norm.reshape(EP // C, C)

    xw1 = _mm_tc(xp, W1)
    p1 = _spmm_sc(xw1.reshape(T * NP, H), row2, col2, nrm2)
    xw2 = _cmb_tc(p1, b1.reshape(1, H), W2)
    p2 = _spmm_sc(xw2.reshape(T * NP, H), row2, col2, nrm2)

    seq = _gru_tc(p2, xp, b2.reshape(1, H), Wih.T, Whh.T,
                  bih.reshape(1, 3 * H), bhh.reshape(1, 3 * H))
    return seq[:, :N, :]


# speedup vs baseline: 26.2837x; 1.2940x over previous
import functools

import jax
import jax.numpy as jnp
from jax import lax
from jax.experimental import pallas as pl
from jax.experimental.pallas import tpu as pltpu
from jax.experimental.pallas import tpu_sc as plsc

N = 10000
E = 320000
T = 8
D = 128
DE = 16
H = 128

NP = 10240
NC = 2
NS = 16
NW = NC * NS
C = 96
EF = E + N
EW = 11520
EP = EW * NW
CH = EW // C
RPT = NP // NS
AR = 10112
RPA = AR // NS

_SC_MESH = plsc.VectorSubcoreMesh(core_axis_name="c", subcore_axis_name="s")
_SC_PARAMS = pltpu.CompilerParams(needs_layout_passes=False)

BN = 2048
NBK = NP // BN


def _deg_body(col_hbm, w_hbm, out_hbm, colbuf, wbuf, acc):
    cid = lax.axis_index("c")
    sid = lax.axis_index("s")
    wid = cid * NS + sid

    def _zero(i, _):
        acc[pl.ds(i * 16, 16)] = jnp.zeros((16,), jnp.float32)
        return 0

    lax.fori_loop(0, NP // 16, _zero, 0)

    def _chunk(g, _):
        base = wid * EW + g * C
        pltpu.sync_copy(col_hbm.at[pl.ds(base, C)], colbuf)
        pltpu.sync_copy(w_hbm.at[pl.ds(base, C)], wbuf)

        def _grp(k, _):
            idx = colbuf[pl.ds(k * 16, 16)]
            val = wbuf[pl.ds(k * 16, 16)]
            plsc.addupdate_scatter(acc, [idx], val)
            return 0

        lax.fori_loop(0, C // 16, _grp, 0)
        return 0

    lax.fori_loop(0, CH, _chunk, 0)
    pltpu.sync_copy(acc, out_hbm.at[wid])


_deg_sc = functools.partial(
    pl.kernel,
    _deg_body,
    out_type=jax.ShapeDtypeStruct((NW, NP), jnp.float32),
    mesh=_SC_MESH,
    scratch_types=[
        pltpu.VMEM((C,), jnp.int32),
        pltpu.VMEM((C,), jnp.float32),
        pltpu.VMEM((NP,), jnp.float32),
    ],
    compiler_params=_SC_PARAMS,
)()


def _norm_body(row_hbm, col_hbm, w_hbm, dinv_hbm, out_hbm,
               rowbuf, colbuf, wbuf, nbuf, dinv_v):
    cid = lax.axis_index("c")
    sid = lax.axis_index("s")
    wid = cid * NS + sid
    pltpu.sync_copy(dinv_hbm, dinv_v)

    def _chunk(g, _):
        base = wid * EW + g * C
        pltpu.sync_copy(row_hbm.at[pl.ds(base, C)], rowbuf)
        pltpu.sync_copy(col_hbm.at[pl.ds(base, C)], colbuf)
        pltpu.sync_copy(w_hbm.at[pl.ds(base, C)], wbuf)

        def _grp(k, _):
            sl = pl.ds(k * 16, 16)
            dr = plsc.load_gather(dinv_v, [rowbuf[sl]])
            dc = plsc.load_gather(dinv_v, [colbuf[sl]])
            nbuf[sl] = dr * wbuf[sl] * dc
            return 0

        lax.fori_loop(0, C // 16, _grp, 0)
        pltpu.sync_copy(nbuf, out_hbm.at[pl.ds(base, C)])
        return 0

    lax.fori_loop(0, CH, _chunk, 0)


_norm_sc = functools.partial(
    pl.kernel,
    _norm_body,
    out_type=jax.ShapeDtypeStruct((EP,), jnp.float32),
    mesh=_SC_MESH,
    scratch_types=[
        pltpu.VMEM((C,), jnp.int32),
        pltpu.VMEM((C,), jnp.int32),
        pltpu.VMEM((C,), jnp.float32),
        pltpu.VMEM((C,), jnp.float32),
        pltpu.VMEM((NP,), jnp.float32),
    ],
    compiler_params=_SC_PARAMS,
)()


BLK = 24
NBL = CH // BLK


def _spmm_body(xw_hbm, row_hbm, col_hbm, norm_hbm, out_hbm,
               rowblk, colblk, nrmblk, idx_a, idx_b, idx_c,
               gbuf_a, gbuf_b, gbuf_c, acc,
               gsem_a, gsem_b, gsem_c, ssem_a, ssem_b, ssem_c):
    cid = lax.axis_index("c")
    sid = lax.axis_index("s")
    wid = cid * NS + sid
    wrow = wid * CH

    idx = (idx_a, idx_b, idx_c)
    gbufs = (gbuf_a, gbuf_b, gbuf_c)
    gsems = (gsem_a, gsem_b, gsem_c)
    ssems = (ssem_a, ssem_b, ssem_c)

    def _load_issue(t, lc, s):
        def _off(k, _):
            sl = pl.ds(k * 16, 16)
            idx[s][sl] = rowblk[lc, sl] + t * NP
            return 0

        lax.fori_loop(0, C // 16, _off, 0)
        pltpu.async_copy(xw_hbm.at[idx[s]], gbufs[s], gsems[s])

    def _scale(lc, s):
        def _grp(k, _):
            nv = nrmblk[lc, pl.ds(k * 16, 16)]
            for l in range(16):
                nsplat = jnp.broadcast_to(nv[l], (16,))
                for j in range(H // 16):
                    sl = pl.ds(j * 16, 16)
                    gbufs[s][k * 16 + l, sl] = gbufs[s][k * 16 + l, sl] * nsplat
            return 0

        lax.fori_loop(0, C // 16, _grp, 0)

    def _wait_gather(s):
        pltpu.make_async_copy(xw_hbm.at[idx[s]], gbufs[s], gsems[s]).wait()

    def _issue_scatter(lc, s):
        pltpu.async_copy(gbufs[s], acc.at[colblk.at[lc]], ssems[s], add=True)

    def _wait_scatter(lc, s):
        pltpu.make_async_copy(gbufs[s], acc.at[colblk.at[lc]], ssems[s]).wait()

    def _step(t, _):
        def _zrow(i, _):
            for j in range(H // 16):
                gbuf_a[i, pl.ds(j * 16, 16)] = jnp.zeros((16,), jnp.float32)
            return 0

        lax.fori_loop(0, C, _zrow, 0)

        def _zcp(i, _):
            pltpu.sync_copy(gbuf_a, acc.at[pl.ds(sid * RPA + i * C, C)])
            return 0

        lax.fori_loop(0, RPA // C, _zcp, 0)
        pltpu.sync_copy(gbuf_a.at[pl.ds(0, RPA - (RPA // C) * C)],
                        acc.at[pl.ds(sid * RPA + (RPA // C) * C,
                                     RPA - (RPA // C) * C)])
        plsc.subcore_barrier()

        def _blk(b, _):
            brow = wrow + b * BLK
            pltpu.sync_copy(row_hbm.at[pl.ds(brow, BLK)], rowblk)
            pltpu.sync_copy(col_hbm.at[pl.ds(brow, BLK)], colblk)
            pltpu.sync_copy(norm_hbm.at[pl.ds(brow, BLK)], nrmblk)
            _load_issue(t, 0, 0)
            _load_issue(t, 1, 1)

            def _tri(q, _):
                for m in range(3):
                    c = 3 * q + m
                    s2 = (m + 2) % 3

                    if m == 0:
                        @pl.when(q >= 1)
                        def _():
                            _wait_scatter(c - 1, s2)

                        _load_issue(t, c + 2, s2)
                    else:
                        _wait_scatter(c - 1, s2)

                        @pl.when(q < BLK // 3 - 1)
                        def _():
                            _load_issue(t, c + 2, s2)

                    _wait_gather(m)
                    _scale(c, m)
                    _issue_scatter(c, m)
                return 0

            lax.fori_loop(0, BLK // 3, _tri, 0)
            _wait_scatter(BLK - 1, 2)
            return 0

        lax.fori_loop(0, NBL, _blk, 0)
        plsc.subcore_barrier()
        pltpu.sync_copy(acc.at[pl.ds(sid * RPA, RPA)],
                        out_hbm.at[cid, t, pl.ds(sid * RPA, RPA)])
        plsc.subcore_barrier()
        return 0

    lax.fori_loop(0, T, _step, 0)


_spmm_sc = functools.partial(
    pl.kernel,
    _spmm_body,
    out_type=jax.ShapeDtypeStruct((NC, T, NP, H), jnp.float32),
    mesh=_SC_MESH,
    scratch_types=[
        pltpu.VMEM((BLK, C), jnp.int32),
        pltpu.VMEM((BLK, C), jnp.int32),
        pltpu.VMEM((BLK, C), jnp.float32),
        pltpu.VMEM((C,), jnp.int32),
        pltpu.VMEM((C,), jnp.int32),
        pltpu.VMEM((C,), jnp.int32),
        pltpu.VMEM((C, H), jnp.float32),
        pltpu.VMEM((C, H), jnp.float32),
        pltpu.VMEM((C, H), jnp.float32),
        pltpu.VMEM_SHARED((AR, H), jnp.float32),
        pltpu.SemaphoreType.DMA,
        pltpu.SemaphoreType.DMA,
        pltpu.SemaphoreType.DMA,
        pltpu.SemaphoreType.DMA,
        pltpu.SemaphoreType.DMA,
        pltpu.SemaphoreType.DMA,
    ],
    compiler_params=_SC_PARAMS,
)()


def _ew_body(ef_ref, web_ref, be_ref, out_ref):
    out_ref[...] = jnp.logaddexp(
        jnp.dot(ef_ref[...], web_ref[...], preferred_element_type=jnp.float32)
        + be_ref[0, 0], 0.0)


def _ew_tc(ef2, We_big, be):
    RB = 4000
    nb = ef2.shape[0] // RB
    return pl.pallas_call(
        _ew_body,
        grid=(nb,),
        in_specs=[
            pl.BlockSpec((RB, 8 * DE), lambda i: (i, 0)),
            pl.BlockSpec((8 * DE, 8), lambda i: (0, 0)),
            pl.BlockSpec((1, 1), lambda i: (0, 0), memory_space=pltpu.SMEM),
        ],
        out_specs=pl.BlockSpec((RB, 8), lambda i: (i, 0)),
        out_shape=jax.ShapeDtypeStruct((ef2.shape[0], 8), jnp.float32),
    )(ef2, We_big, be)


def _dinv_body(dp_ref, out_ref):
    deg = jnp.sum(dp_ref[...], axis=0, keepdims=True)
    out_ref[...] = jnp.where(deg > 0, jax.lax.rsqrt(deg), 0.0)


def _dinv_tc(deg_part):
    return pl.pallas_call(
        _dinv_body,
        grid=(NBK,),
        in_specs=[pl.BlockSpec((NW, BN), lambda i: (0, i))],
        out_specs=pl.BlockSpec((1, BN), lambda i: (0, i)),
        out_shape=jax.ShapeDtypeStruct((1, NP), jnp.float32),
    )(deg_part)


def _mm_body(x_ref, w_ref, out_ref):
    out_ref[0] = jnp.dot(x_ref[0], w_ref[...], preferred_element_type=jnp.float32)


def _mm_tc(xp, W):
    return pl.pallas_call(
        _mm_body,
        grid=(T, NBK),
        in_specs=[
            pl.BlockSpec((1, BN, H), lambda t, nb: (t, nb, 0)),
            pl.BlockSpec((H, H), lambda t, nb: (0, 0)),
        ],
        out_specs=pl.BlockSpec((1, BN, H), lambda t, nb: (t, nb, 0)),
        out_shape=jax.ShapeDtypeStruct((T, NP, H), jnp.float32),
    )(xp, W)


def _cmb_body(p_ref, b_ref, w_ref, out_ref):
    h1 = jnp.tanh(p_ref[0, 0] + p_ref[1, 0] + b_ref[...])
    out_ref[0] = jnp.dot(h1, w_ref[...], preferred_element_type=jnp.float32)


def _cmb_tc(parts, b1, W2):
    return pl.pallas_call(
        _cmb_body,
        grid=(T, NBK),
        in_specs=[
            pl.BlockSpec((NC, 1, BN, H), lambda t, nb: (0, t, nb, 0)),
            pl.BlockSpec((1, H), lambda t, nb: (0, 0)),
            pl.BlockSpec((H, H), lambda t, nb: (0, 0)),
        ],
        out_specs=pl.BlockSpec((1, BN, H), lambda t, nb: (t, nb, 0)),
        out_shape=jax.ShapeDtypeStruct((T, NP, H), jnp.float32),
    )(parts, b1, W2)


def _gru_body(p_ref, x_ref, b_ref, wih_ref, whh_ref, bih_ref, bhh_ref,
              out_ref, h_ref):
    t = pl.program_id(0)
    nb = pl.program_id(1)
    g = p_ref[0, 0] + p_ref[1, 0] + b_ref[...] + x_ref[0]
    h = jnp.where(t == 0, jnp.zeros_like(h_ref[nb]), h_ref[nb])
    gi = jnp.dot(g, wih_ref[...], preferred_element_type=jnp.float32) + bih_ref[...]
    gh = jnp.dot(h, whh_ref[...], preferred_element_type=jnp.float32) + bhh_ref[...]
    i_r, i_z, i_n = gi[:, :H], gi[:, H:2 * H], gi[:, 2 * H:]
    h_r, h_z, h_n = gh[:, :H], gh[:, H:2 * H], gh[:, 2 * H:]
    r = jax.nn.sigmoid(i_r + h_r)
    z = jax.nn.sigmoid(i_z + h_z)
    n = jnp.tanh(i_n + r * h_n)
    hn = (1.0 - z) * n + z * h
    h_ref[nb] = hn
    out_ref[0] = hn


def _gru_tc(parts, xp, b2, WihT, WhhT, bih, bhh):
    return pl.pallas_call(
        _gru_body,
        grid=(T, NBK),
        in_specs=[
            pl.BlockSpec((NC, 1, BN, H), lambda t, nb: (0, t, nb, 0)),
            pl.BlockSpec((1, BN, H), lambda t, nb: (t, nb, 0)),
            pl.BlockSpec((1, H), lambda t, nb: (0, 0)),
            pl.BlockSpec((H, 3 * H), lambda t, nb: (0, 0)),
            pl.BlockSpec((H, 3 * H), lambda t, nb: (0, 0)),
            pl.BlockSpec((1, 3 * H), lambda t, nb: (0, 0)),
            pl.BlockSpec((1, 3 * H), lambda t, nb: (0, 0)),
        ],
        out_specs=pl.BlockSpec((1, BN, H), lambda t, nb: (t, nb, 0)),
        out_shape=jax.ShapeDtypeStruct((T, NP, H), jnp.float32),
        scratch_shapes=[pltpu.VMEM((NBK, BN, H), jnp.float32)],
    )(parts, xp, b2, WihT, WhhT, bih, bhh)


def kernel(edge_index, edge_feats, node_feats, We, be, W1, b1, W2, b2, Wih, Whh, bih, bhh):
    row = edge_index[0]
    col = edge_index[1]

    We_big = jnp.kron(jnp.eye(8, dtype=We.dtype), We)
    ef2 = edge_feats.reshape(E // 8, 8 * DE)
    w = _ew_tc(ef2, We_big, be.reshape(1, 1)).reshape(E)

    nodes = jnp.arange(N, dtype=row.dtype)
    pad = EP - EF
    spread = jnp.arange(pad, dtype=row.dtype) % N
    row_f = jnp.concatenate([row, nodes, spread])
    col_f = jnp.concatenate([col, nodes, spread])
    w_f = jnp.concatenate([w, jnp.ones((N,), w.dtype), jnp.zeros((pad,), w.dtype)])

    deg_part = _deg_sc(col_f, w_f)
    dinv = _dinv_tc(deg_part).reshape(NP)
    norm = _norm_sc(row_f, col_f, w_f, dinv)

    xp = jnp.pad(node_feats, ((0, 0), (0, NP - N), (0, 0)))

    row2 = row_f.reshape(EP // C, C)
    col2 = col_f.reshape(EP // C, C)
    nrm2 = norm.reshape(EP // C, C)

    xw1 = _mm_tc(xp, W1)
    p1 = _spmm_sc(xw1.reshape(T * NP, H), row2, col2, nrm2)
    xw2 = _cmb_tc(p1, b1.reshape(1, H), W2)
    p2 = _spmm_sc(xw2.reshape(T * NP, H), row2, col2, nrm2)

    seq = _gru_tc(p2, xp, b2.reshape(1, H), Wih.T, Whh.T,
                  bih.reshape(1, 3 * H), bhh.reshape(1, 3 * H))
    return seq[:, :N, :]


# speedup vs baseline: 28.1950x; 1.0727x over previous
import functools

import jax
import jax.numpy as jnp
from jax import lax
from jax.experimental import pallas as pl
from jax.experimental.pallas import tpu as pltpu
from jax.experimental.pallas import tpu_sc as plsc

N = 10000
E = 320000
T = 8
D = 128
DE = 16
H = 128

NP = 10240
NC = 2
NS = 16
NW = NC * NS
C = 96
EF = E + N
EW = 11520
EP = EW * NW
CH = EW // C
RPT = NP // NS
AR = 10112
RPA = AR // NS

_SC_MESH = plsc.VectorSubcoreMesh(core_axis_name="c", subcore_axis_name="s")
_SC_PARAMS = pltpu.CompilerParams(needs_layout_passes=False)

BN = 2048
NBK = NP // BN


CB = 1152
NCB = EW // CB


def _deg_body(col_hbm, w_hbm, out_hbm, colbuf, wbuf, acc):
    cid = lax.axis_index("c")
    sid = lax.axis_index("s")
    wid = cid * NS + sid

    def _zero(i, _):
        acc[pl.ds(i * 16, 16)] = jnp.zeros((16,), jnp.float32)
        return 0

    lax.fori_loop(0, NP // 16, _zero, 0)

    def _blk(g, _):
        base = wid * EW + g * CB
        pltpu.sync_copy(col_hbm.at[pl.ds(base, CB)], colbuf)
        pltpu.sync_copy(w_hbm.at[pl.ds(base, CB)], wbuf)

        def _grp(k, _):
            idx = colbuf[pl.ds(k * 16, 16)]
            val = wbuf[pl.ds(k * 16, 16)]
            plsc.addupdate_scatter(acc, [idx], val)
            return 0

        lax.fori_loop(0, CB // 16, _grp, 0)
        return 0

    lax.fori_loop(0, NCB, _blk, 0)
    pltpu.sync_copy(acc, out_hbm.at[wid])


_deg_sc = functools.partial(
    pl.kernel,
    _deg_body,
    out_type=jax.ShapeDtypeStruct((NW, NP), jnp.float32),
    mesh=_SC_MESH,
    scratch_types=[
        pltpu.VMEM((CB,), jnp.int32),
        pltpu.VMEM((CB,), jnp.float32),
        pltpu.VMEM((NP,), jnp.float32),
    ],
    compiler_params=_SC_PARAMS,
)()


def _norm_body(row_hbm, col_hbm, w_hbm, dinv_hbm, out_hbm,
               rowbuf, colbuf, wbuf, nbuf, dinv_v):
    cid = lax.axis_index("c")
    sid = lax.axis_index("s")
    wid = cid * NS + sid
    pltpu.sync_copy(dinv_hbm, dinv_v)

    def _blk(g, _):
        base = wid * EW + g * CB
        pltpu.sync_copy(row_hbm.at[pl.ds(base, CB)], rowbuf)
        pltpu.sync_copy(col_hbm.at[pl.ds(base, CB)], colbuf)
        pltpu.sync_copy(w_hbm.at[pl.ds(base, CB)], wbuf)

        def _grp(k, _):
            sl = pl.ds(k * 16, 16)
            dr = plsc.load_gather(dinv_v, [rowbuf[sl]])
            dc = plsc.load_gather(dinv_v, [colbuf[sl]])
            nbuf[sl] = dr * wbuf[sl] * dc
            return 0

        lax.fori_loop(0, CB // 16, _grp, 0)
        pltpu.sync_copy(nbuf, out_hbm.at[pl.ds(base, CB)])
        return 0

    lax.fori_loop(0, NCB, _blk, 0)


_norm_sc = functools.partial(
    pl.kernel,
    _norm_body,
    out_type=jax.ShapeDtypeStruct((EP,), jnp.float32),
    mesh=_SC_MESH,
    scratch_types=[
        pltpu.VMEM((CB,), jnp.int32),
        pltpu.VMEM((CB,), jnp.int32),
        pltpu.VMEM((CB,), jnp.float32),
        pltpu.VMEM((CB,), jnp.float32),
        pltpu.VMEM((NP,), jnp.float32),
    ],
    compiler_params=_SC_PARAMS,
)()


BLK = 24
NBL = CH // BLK


def _spmm_body(xw_hbm, row_hbm, col_hbm, norm_hbm, out_hbm,
               rowblk, colblk, nrmblk, idx_a, idx_b, idx_c,
               gbuf_a, gbuf_b, gbuf_c, acc,
               gsem_a, gsem_b, gsem_c, ssem_a, ssem_b, ssem_c):
    cid = lax.axis_index("c")
    sid = lax.axis_index("s")
    wid = cid * NS + sid
    wrow = wid * CH

    idx = (idx_a, idx_b, idx_c)
    gbufs = (gbuf_a, gbuf_b, gbuf_c)
    gsems = (gsem_a, gsem_b, gsem_c)
    ssems = (ssem_a, ssem_b, ssem_c)

    def _load_issue(t, lc, s):
        def _off(k, _):
            sl = pl.ds(k * 16, 16)
            idx[s][sl] = rowblk[lc, sl] + t * NP
            return 0

        lax.fori_loop(0, C // 16, _off, 0)
        pltpu.async_copy(xw_hbm.at[idx[s]], gbufs[s], gsems[s])

    def _scale(lc, s):
        def _grp(k, _):
            nv = nrmblk[lc, pl.ds(k * 16, 16)]
            for l in range(16):
                nsplat = jnp.broadcast_to(nv[l], (16,))
                for j in range(H // 16):
                    sl = pl.ds(j * 16, 16)
                    gbufs[s][k * 16 + l, sl] = gbufs[s][k * 16 + l, sl] * nsplat
            return 0

        lax.fori_loop(0, C // 16, _grp, 0)

    def _wait_gather(s):
        pltpu.make_async_copy(xw_hbm.at[idx[s]], gbufs[s], gsems[s]).wait()

    def _issue_scatter(lc, s):
        pltpu.async_copy(gbufs[s], acc.at[colblk.at[lc]], ssems[s], add=True)

    def _wait_scatter(lc, s):
        pltpu.make_async_copy(gbufs[s], acc.at[colblk.at[lc]], ssems[s]).wait()

    def _step(t, _):
        def _zrow(i, _):
            for j in range(H // 16):
                gbuf_a[i, pl.ds(j * 16, 16)] = jnp.zeros((16,), jnp.float32)
            return 0

        lax.fori_loop(0, C, _zrow, 0)

        def _zcp(i, _):
            pltpu.sync_copy(gbuf_a, acc.at[pl.ds(sid * RPA + i * C, C)])
            return 0

        lax.fori_loop(0, RPA // C, _zcp, 0)
        pltpu.sync_copy(gbuf_a.at[pl.ds(0, RPA - (RPA // C) * C)],
                        acc.at[pl.ds(sid * RPA + (RPA // C) * C,
                                     RPA - (RPA // C) * C)])
        plsc.subcore_barrier()

        def _blk(b, _):
            brow = wrow + b * BLK
            pltpu.sync_copy(row_hbm.at[pl.ds(brow, BLK)], rowblk)
            pltpu.sync_copy(col_hbm.at[pl.ds(brow, BLK)], colblk)
            pltpu.sync_copy(norm_hbm.at[pl.ds(brow, BLK)], nrmblk)
            _load_issue(t, 0, 0)
            _load_issue(t, 1, 1)

            def _tri(q, _):
                for m in range(3):
                    c = 3 * q + m
                    s2 = (m + 2) % 3

                    if m == 0:
                        @pl.when(q >= 1)
                        def _():
                            _wait_scatter(c - 1, s2)

                        _load_issue(t, c + 2, s2)
                    else:
                        _wait_scatter(c - 1, s2)

                        @pl.when(q < BLK // 3 - 1)
                        def _():
                            _load_issue(t, c + 2, s2)

                    _wait_gather(m)
                    _scale(c, m)
                    _issue_scatter(c, m)
                return 0

            lax.fori_loop(0, BLK // 3, _tri, 0)
            _wait_scatter(BLK - 1, 2)
            return 0

        lax.fori_loop(0, NBL, _blk, 0)
        plsc.subcore_barrier()
        pltpu.sync_copy(acc.at[pl.ds(sid * RPA, RPA)],
                        out_hbm.at[cid, t, pl.ds(sid * RPA, RPA)])
        plsc.subcore_barrier()
        return 0

    lax.fori_loop(0, T, _step, 0)


_spmm_sc = functools.partial(
    pl.kernel,
    _spmm_body,
    out_type=jax.ShapeDtypeStruct((NC, T, NP, H), jnp.float32),
    mesh=_SC_MESH,
    scratch_types=[
        pltpu.VMEM((BLK, C), jnp.int32),
        pltpu.VMEM((BLK, C), jnp.int32),
        pltpu.VMEM((BLK, C), jnp.float32),
        pltpu.VMEM((C,), jnp.int32),
        pltpu.VMEM((C,), jnp.int32),
        pltpu.VMEM((C,), jnp.int32),
        pltpu.VMEM((C, H), jnp.float32),
        pltpu.VMEM((C, H), jnp.float32),
        pltpu.VMEM((C, H), jnp.float32),
        pltpu.VMEM_SHARED((AR, H), jnp.float32),
        pltpu.SemaphoreType.DMA,
        pltpu.SemaphoreType.DMA,
        pltpu.SemaphoreType.DMA,
        pltpu.SemaphoreType.DMA,
        pltpu.SemaphoreType.DMA,
        pltpu.SemaphoreType.DMA,
    ],
    compiler_params=_SC_PARAMS,
)()


def _ew_body(ef_ref, web_ref, be_ref, out_ref):
    out_ref[...] = jnp.logaddexp(
        jnp.dot(ef_ref[...], web_ref[...], preferred_element_type=jnp.float32)
        + be_ref[0, 0], 0.0)


def _ew_tc(ef2, We_big, be):
    RB = 4000
    nb = ef2.shape[0] // RB
    return pl.pallas_call(
        _ew_body,
        grid=(nb,),
        in_specs=[
            pl.BlockSpec((RB, 8 * DE), lambda i: (i, 0)),
            pl.BlockSpec((8 * DE, 8), lambda i: (0, 0)),
            pl.BlockSpec((1, 1), lambda i: (0, 0), memory_space=pltpu.SMEM),
        ],
        out_specs=pl.BlockSpec((RB, 8), lambda i: (i, 0)),
        out_shape=jax.ShapeDtypeStruct((ef2.shape[0], 8), jnp.float32),
    )(ef2, We_big, be)


def _dinv_body(dp_ref, out_ref):
    deg = jnp.sum(dp_ref[...], axis=0, keepdims=True)
    out_ref[...] = jnp.where(deg > 0, jax.lax.rsqrt(deg), 0.0)


def _dinv_tc(deg_part):
    return pl.pallas_call(
        _dinv_body,
        grid=(NBK,),
        in_specs=[pl.BlockSpec((NW, BN), lambda i: (0, i))],
        out_specs=pl.BlockSpec((1, BN), lambda i: (0, i)),
        out_shape=jax.ShapeDtypeStruct((1, NP), jnp.float32),
    )(deg_part)


def _mm_body(x_ref, w_ref, out_ref):
    out_ref[0] = jnp.dot(x_ref[0], w_ref[...], preferred_element_type=jnp.float32)


def _mm_tc(xp, W):
    return pl.pallas_call(
        _mm_body,
        grid=(T, NBK),
        in_specs=[
            pl.BlockSpec((1, BN, H), lambda t, nb: (t, nb, 0)),
            pl.BlockSpec((H, H), lambda t, nb: (0, 0)),
        ],
        out_specs=pl.BlockSpec((1, BN, H), lambda t, nb: (t, nb, 0)),
        out_shape=jax.ShapeDtypeStruct((T, NP, H), jnp.float32),
    )(xp, W)


def _cmb_body(p_ref, b_ref, w_ref, out_ref):
    h1 = jnp.tanh(p_ref[0, 0] + p_ref[1, 0] + b_ref[...])
    out_ref[0] = jnp.dot(h1, w_ref[...], preferred_element_type=jnp.float32)


def _cmb_tc(parts, b1, W2):
    return pl.pallas_call(
        _cmb_body,
        grid=(T, NBK),
        in_specs=[
            pl.BlockSpec((NC, 1, BN, H), lambda t, nb: (0, t, nb, 0)),
            pl.BlockSpec((1, H), lambda t, nb: (0, 0)),
            pl.BlockSpec((H, H), lambda t, nb: (0, 0)),
        ],
        out_specs=pl.BlockSpec((1, BN, H), lambda t, nb: (t, nb, 0)),
        out_shape=jax.ShapeDtypeStruct((T, NP, H), jnp.float32),
    )(parts, b1, W2)


def _gru_body(p_ref, x_ref, b_ref, wih_ref, whh_ref, bih_ref, bhh_ref,
              out_ref, h_ref):
    t = pl.program_id(0)
    nb = pl.program_id(1)
    g = p_ref[0, 0] + p_ref[1, 0] + b_ref[...] + x_ref[0]
    h = jnp.where(t == 0, jnp.zeros_like(h_ref[nb]), h_ref[nb])
    gi = jnp.dot(g, wih_ref[...], preferred_element_type=jnp.float32) + bih_ref[...]
    gh = jnp.dot(h, whh_ref[...], preferred_element_type=jnp.float32) + bhh_ref[...]
    i_r, i_z, i_n = gi[:, :H], gi[:, H:2 * H], gi[:, 2 * H:]
    h_r, h_z, h_n = gh[:, :H], gh[:, H:2 * H], gh[:, 2 * H:]
    r = jax.nn.sigmoid(i_r + h_r)
    z = jax.nn.sigmoid(i_z + h_z)
    n = jnp.tanh(i_n + r * h_n)
    hn = (1.0 - z) * n + z * h
    h_ref[nb] = hn
    out_ref[0] = hn


def _gru_tc(parts, xp, b2, WihT, WhhT, bih, bhh):
    return pl.pallas_call(
        _gru_body,
        grid=(T, NBK),
        in_specs=[
            pl.BlockSpec((NC, 1, BN, H), lambda t, nb: (0, t, nb, 0)),
            pl.BlockSpec((1, BN, H), lambda t, nb: (t, nb, 0)),
            pl.BlockSpec((1, H), lambda t, nb: (0, 0)),
            pl.BlockSpec((H, 3 * H), lambda t, nb: (0, 0)),
            pl.BlockSpec((H, 3 * H), lambda t, nb: (0, 0)),
            pl.BlockSpec((1, 3 * H), lambda t, nb: (0, 0)),
            pl.BlockSpec((1, 3 * H), lambda t, nb: (0, 0)),
        ],
        out_specs=pl.BlockSpec((1, BN, H), lambda t, nb: (t, nb, 0)),
        out_shape=jax.ShapeDtypeStruct((T, NP, H), jnp.float32),
        scratch_shapes=[pltpu.VMEM((NBK, BN, H), jnp.float32)],
    )(parts, xp, b2, WihT, WhhT, bih, bhh)


def kernel(edge_index, edge_feats, node_feats, We, be, W1, b1, W2, b2, Wih, Whh, bih, bhh):
    row = edge_index[0]
    col = edge_index[1]

    We_big = jnp.kron(jnp.eye(8, dtype=We.dtype), We)
    ef2 = edge_feats.reshape(E // 8, 8 * DE)
    w = _ew_tc(ef2, We_big, be.reshape(1, 1)).reshape(E)

    nodes = jnp.arange(N, dtype=row.dtype)
    pad = EP - EF
    spread = jnp.arange(pad, dtype=row.dtype) % N
    row_f = jnp.concatenate([row, nodes, spread])
    col_f = jnp.concatenate([col, nodes, spread])
    w_f = jnp.concatenate([w, jnp.ones((N,), w.dtype), jnp.zeros((pad,), w.dtype)])

    deg_part = _deg_sc(col_f, w_f)
    dinv = _dinv_tc(deg_part).reshape(NP)
    norm = _norm_sc(row_f, col_f, w_f, dinv)

    xp = jnp.pad(node_feats, ((0, 0), (0, NP - N), (0, 0)))

    row2 = row_f.reshape(EP // C, C)
    col2 = col_f.reshape(EP // C, C)
    nrm2 = norm.reshape(EP // C, C)

    xw1 = _mm_tc(xp, W1)
    p1 = _spmm_sc(xw1.reshape(T * NP, H), row2, col2, nrm2)
    xw2 = _cmb_tc(p1, b1.reshape(1, H), W2)
    p2 = _spmm_sc(xw2.reshape(T * NP, H), row2, col2, nrm2)

    seq = _gru_tc(p2, xp, b2.reshape(1, H), Wih.T, Whh.T,
                  bih.reshape(1, 3 * H), bhh.reshape(1, 3 * H))
    return seq[:, :N, :]


# speedup vs baseline: 29.5338x; 1.0475x over previous
import functools

import jax
import jax.numpy as jnp
from jax import lax
from jax.experimental import pallas as pl
from jax.experimental.pallas import tpu as pltpu
from jax.experimental.pallas import tpu_sc as plsc

N = 10000
E = 320000
T = 8
D = 128
DE = 16
H = 128

NP = 10240
NC = 2
NS = 16
NW = NC * NS
C = 96
EF = E + N
EW = 11520
EP = EW * NW
CH = EW // C
RPT = NP // NS
AR = 10112
RPA = AR // NS

_SC_MESH = plsc.VectorSubcoreMesh(core_axis_name="c", subcore_axis_name="s")
_SC_PARAMS = pltpu.CompilerParams(needs_layout_passes=False)

BN = 2048
NBK = NP // BN


CB = 1152
NCB = EW // CB


def _deg_body(col_hbm, w_hbm, out_hbm, colbuf, wbuf, acc):
    cid = lax.axis_index("c")
    sid = lax.axis_index("s")
    wid = cid * NS + sid

    def _zero(i, _):
        acc[pl.ds(i * 16, 16)] = jnp.zeros((16,), jnp.float32)
        return 0

    lax.fori_loop(0, NP // 16, _zero, 0)

    def _blk(g, _):
        base = wid * EW + g * CB
        pltpu.sync_copy(col_hbm.at[pl.ds(base, CB)], colbuf)
        pltpu.sync_copy(w_hbm.at[pl.ds(base, CB)], wbuf)

        def _grp(k, _):
            idx = colbuf[pl.ds(k * 16, 16)]
            val = wbuf[pl.ds(k * 16, 16)]
            plsc.addupdate_scatter(acc, [idx], val)
            return 0

        lax.fori_loop(0, CB // 16, _grp, 0)
        return 0

    lax.fori_loop(0, NCB, _blk, 0)
    pltpu.sync_copy(acc, out_hbm.at[wid])


_deg_sc = functools.partial(
    pl.kernel,
    _deg_body,
    out_type=jax.ShapeDtypeStruct((NW, NP), jnp.float32),
    mesh=_SC_MESH,
    scratch_types=[
        pltpu.VMEM((CB,), jnp.int32),
        pltpu.VMEM((CB,), jnp.float32),
        pltpu.VMEM((NP,), jnp.float32),
    ],
    compiler_params=_SC_PARAMS,
)()


def _norm_body(row_hbm, col_hbm, w_hbm, dinv_hbm, out_hbm,
               rowbuf, colbuf, wbuf, nbuf, dinv_v):
    cid = lax.axis_index("c")
    sid = lax.axis_index("s")
    wid = cid * NS + sid
    pltpu.sync_copy(dinv_hbm, dinv_v)

    def _blk(g, _):
        base = wid * EW + g * CB
        pltpu.sync_copy(row_hbm.at[pl.ds(base, CB)], rowbuf)
        pltpu.sync_copy(col_hbm.at[pl.ds(base, CB)], colbuf)
        pltpu.sync_copy(w_hbm.at[pl.ds(base, CB)], wbuf)

        def _grp(k, _):
            sl = pl.ds(k * 16, 16)
            dr = plsc.load_gather(dinv_v, [rowbuf[sl]])
            dc = plsc.load_gather(dinv_v, [colbuf[sl]])
            nbuf[sl] = dr * wbuf[sl] * dc
            return 0

        lax.fori_loop(0, CB // 16, _grp, 0)
        pltpu.sync_copy(nbuf, out_hbm.at[pl.ds(base, CB)])
        return 0

    lax.fori_loop(0, NCB, _blk, 0)


_norm_sc = functools.partial(
    pl.kernel,
    _norm_body,
    out_type=jax.ShapeDtypeStruct((EP,), jnp.float32),
    mesh=_SC_MESH,
    scratch_types=[
        pltpu.VMEM((CB,), jnp.int32),
        pltpu.VMEM((CB,), jnp.int32),
        pltpu.VMEM((CB,), jnp.float32),
        pltpu.VMEM((CB,), jnp.float32),
        pltpu.VMEM((NP,), jnp.float32),
    ],
    compiler_params=_SC_PARAMS,
)()


BLK = 24
NBL = CH // BLK


def _spmm_body(xw_hbm, row_hbm, col_hbm, norm_hbm, out_hbm,
               rowblk, colblk, nrmblk, idx_a, idx_b, idx_c,
               gbuf_a, gbuf_b, gbuf_c, acc,
               gsem_a, gsem_b, gsem_c, ssem_a, ssem_b, ssem_c):
    cid = lax.axis_index("c")
    sid = lax.axis_index("s")
    wid = cid * NS + sid
    wrow = wid * CH

    idx = (idx_a, idx_b, idx_c)
    gbufs = (gbuf_a, gbuf_b, gbuf_c)
    gsems = (gsem_a, gsem_b, gsem_c)
    ssems = (ssem_a, ssem_b, ssem_c)

    def _load_issue(t, lc, s):
        def _off(k, _):
            sl = pl.ds(k * 16, 16)
            idx[s][sl] = rowblk[lc, sl] + t * NP
            return 0

        lax.fori_loop(0, C // 16, _off, 0)
        pltpu.async_copy(xw_hbm.at[idx[s]], gbufs[s], gsems[s])

    def _scale(lc, s):
        def _grp(k, _):
            nv = nrmblk[lc, pl.ds(k * 16, 16)]
            for l in range(16):
                nsplat = jnp.broadcast_to(nv[l], (16,))
                for j in range(H // 16):
                    sl = pl.ds(j * 16, 16)
                    gbufs[s][k * 16 + l, sl] = gbufs[s][k * 16 + l, sl] * nsplat
            return 0

        lax.fori_loop(0, C // 16, _grp, 0)

    def _wait_gather(s):
        pltpu.make_async_copy(xw_hbm.at[idx[s]], gbufs[s], gsems[s]).wait()

    def _issue_scatter(lc, s):
        pltpu.async_copy(gbufs[s], acc.at[colblk.at[lc]], ssems[s], add=True)

    def _wait_scatter(lc, s):
        pltpu.make_async_copy(gbufs[s], acc.at[colblk.at[lc]], ssems[s]).wait()

    def _step(t, _):
        def _zrow(i, _):
            for j in range(H // 16):
                gbuf_a[i, pl.ds(j * 16, 16)] = jnp.zeros((16,), jnp.float32)
            return 0

        lax.fori_loop(0, C, _zrow, 0)

        def _zcp(i, _):
            pltpu.sync_copy(gbuf_a, acc.at[pl.ds(sid * RPA + i * C, C)])
            return 0

        lax.fori_loop(0, RPA // C, _zcp, 0)
        pltpu.sync_copy(gbuf_a.at[pl.ds(0, RPA - (RPA // C) * C)],
                        acc.at[pl.ds(sid * RPA + (RPA // C) * C,
                                     RPA - (RPA // C) * C)])
        plsc.subcore_barrier()

        def _blk(b, _):
            brow = wrow + b * BLK
            pltpu.sync_copy(row_hbm.at[pl.ds(brow, BLK)], rowblk)
            pltpu.sync_copy(col_hbm.at[pl.ds(brow, BLK)], colblk)
            pltpu.sync_copy(norm_hbm.at[pl.ds(brow, BLK)], nrmblk)
            _load_issue(t, 0, 0)
            _load_issue(t, 1, 1)

            def _tri(q, _):
                for m in range(3):
                    c = 3 * q + m
                    s2 = (m + 2) % 3

                    _wait_gather(m)
                    _scale(c, m)
                    _issue_scatter(c, m)

                    if m == 0:
                        @pl.when(q >= 1)
                        def _():
                            _wait_scatter(c - 1, s2)

                        _load_issue(t, c + 2, s2)
                    else:
                        _wait_scatter(c - 1, s2)

                        @pl.when(q < BLK // 3 - 1)
                        def _():
                            _load_issue(t, c + 2, s2)
                return 0

            lax.fori_loop(0, BLK // 3, _tri, 0)
            _wait_scatter(BLK - 1, 2)
            return 0

        lax.fori_loop(0, NBL, _blk, 0)
        plsc.subcore_barrier()
        pltpu.sync_copy(acc.at[pl.ds(sid * RPA, RPA)],
                        out_hbm.at[cid, t, pl.ds(sid * RPA, RPA)])
        plsc.subcore_barrier()
        return 0

    lax.fori_loop(0, T, _step, 0)


_spmm_sc = functools.partial(
    pl.kernel,
    _spmm_body,
    out_type=jax.ShapeDtypeStruct((NC, T, NP, H), jnp.float32),
    mesh=_SC_MESH,
    scratch_types=[
        pltpu.VMEM((BLK, C), jnp.int32),
        pltpu.VMEM((BLK, C), jnp.int32),
        pltpu.VMEM((BLK, C), jnp.float32),
        pltpu.VMEM((C,), jnp.int32),
        pltpu.VMEM((C,), jnp.int32),
        pltpu.VMEM((C,), jnp.int32),
        pltpu.VMEM((C, H), jnp.float32),
        pltpu.VMEM((C, H), jnp.float32),
        pltpu.VMEM((C, H), jnp.float32),
        pltpu.VMEM_SHARED((AR, H), jnp.float32),
        pltpu.SemaphoreType.DMA,
        pltpu.SemaphoreType.DMA,
        pltpu.SemaphoreType.DMA,
        pltpu.SemaphoreType.DMA,
        pltpu.SemaphoreType.DMA,
        pltpu.SemaphoreType.DMA,
    ],
    compiler_params=_SC_PARAMS,
)()


def _ew_body(ef_ref, web_ref, be_ref, out_ref):
    out_ref[...] = jnp.logaddexp(
        jnp.dot(ef_ref[...], web_ref[...], preferred_element_type=jnp.float32)
        + be_ref[0, 0], 0.0)


def _ew_tc(ef2, We_big, be):
    RB = 4000
    nb = ef2.shape[0] // RB
    return pl.pallas_call(
        _ew_body,
        grid=(nb,),
        in_specs=[
            pl.BlockSpec((RB, 8 * DE), lambda i: (i, 0)),
            pl.BlockSpec((8 * DE, 8), lambda i: (0, 0)),
            pl.BlockSpec((1, 1), lambda i: (0, 0), memory_space=pltpu.SMEM),
        ],
        out_specs=pl.BlockSpec((RB, 8), lambda i: (i, 0)),
        out_shape=jax.ShapeDtypeStruct((ef2.shape[0], 8), jnp.float32),
    )(ef2, We_big, be)


def _dinv_body(dp_ref, out_ref):
    deg = jnp.sum(dp_ref[...], axis=0, keepdims=True)
    out_ref[...] = jnp.where(deg > 0, jax.lax.rsqrt(deg), 0.0)


def _dinv_tc(deg_part):
    return pl.pallas_call(
        _dinv_body,
        grid=(NBK,),
        in_specs=[pl.BlockSpec((NW, BN), lambda i: (0, i))],
        out_specs=pl.BlockSpec((1, BN), lambda i: (0, i)),
        out_shape=jax.ShapeDtypeStruct((1, NP), jnp.float32),
    )(deg_part)


def _mm_body(x_ref, w_ref, out_ref):
    out_ref[0] = jnp.dot(x_ref[0], w_ref[...], preferred_element_type=jnp.float32)


def _mm_tc(xp, W):
    return pl.pallas_call(
        _mm_body,
        grid=(T, NBK),
        in_specs=[
            pl.BlockSpec((1, BN, H), lambda t, nb: (t, nb, 0)),
            pl.BlockSpec((H, H), lambda t, nb: (0, 0)),
        ],
        out_specs=pl.BlockSpec((1, BN, H), lambda t, nb: (t, nb, 0)),
        out_shape=jax.ShapeDtypeStruct((T, NP, H), jnp.float32),
    )(xp, W)


def _cmb_body(p_ref, b_ref, w_ref, out_ref):
    h1 = jnp.tanh(p_ref[0, 0] + p_ref[1, 0] + b_ref[...])
    out_ref[0] = jnp.dot(h1, w_ref[...], preferred_element_type=jnp.float32)


def _cmb_tc(parts, b1, W2):
    return pl.pallas_call(
        _cmb_body,
        grid=(T, NBK),
        in_specs=[
            pl.BlockSpec((NC, 1, BN, H), lambda t, nb: (0, t, nb, 0)),
            pl.BlockSpec((1, H), lambda t, nb: (0, 0)),
            pl.BlockSpec((H, H), lambda t, nb: (0, 0)),
        ],
        out_specs=pl.BlockSpec((1, BN, H), lambda t, nb: (t, nb, 0)),
        out_shape=jax.ShapeDtypeStruct((T, NP, H), jnp.float32),
    )(parts, b1, W2)


def _gru_body(p_ref, x_ref, b_ref, wih_ref, whh_ref, bih_ref, bhh_ref,
              out_ref, h_ref):
    t = pl.program_id(0)
    nb = pl.program_id(1)
    g = p_ref[0, 0] + p_ref[1, 0] + b_ref[...] + x_ref[0]
    h = jnp.where(t == 0, jnp.zeros_like(h_ref[nb]), h_ref[nb])
    gi = jnp.dot(g, wih_ref[...], preferred_element_type=jnp.float32) + bih_ref[...]
    gh = jnp.dot(h, whh_ref[...], preferred_element_type=jnp.float32) + bhh_ref[...]
    i_r, i_z, i_n = gi[:, :H], gi[:, H:2 * H], gi[:, 2 * H:]
    h_r, h_z, h_n = gh[:, :H], gh[:, H:2 * H], gh[:, 2 * H:]
    r = jax.nn.sigmoid(i_r + h_r)
    z = jax.nn.sigmoid(i_z + h_z)
    n = jnp.tanh(i_n + r * h_n)
    hn = (1.0 - z) * n + z * h
    h_ref[nb] = hn
    out_ref[0] = hn


def _gru_tc(parts, xp, b2, WihT, WhhT, bih, bhh):
    return pl.pallas_call(
        _gru_body,
        grid=(T, NBK),
        in_specs=[
            pl.BlockSpec((NC, 1, BN, H), lambda t, nb: (0, t, nb, 0)),
            pl.BlockSpec((1, BN, H), lambda t, nb: (t, nb, 0)),
            pl.BlockSpec((1, H), lambda t, nb: (0, 0)),
            pl.BlockSpec((H, 3 * H), lambda t, nb: (0, 0)),
            pl.BlockSpec((H, 3 * H), lambda t, nb: (0, 0)),
            pl.BlockSpec((1, 3 * H), lambda t, nb: (0, 0)),
            pl.BlockSpec((1, 3 * H), lambda t, nb: (0, 0)),
        ],
        out_specs=pl.BlockSpec((1, BN, H), lambda t, nb: (t, nb, 0)),
        out_shape=jax.ShapeDtypeStruct((T, NP, H), jnp.float32),
        scratch_shapes=[pltpu.VMEM((NBK, BN, H), jnp.float32)],
    )(parts, xp, b2, WihT, WhhT, bih, bhh)


def kernel(edge_index, edge_feats, node_feats, We, be, W1, b1, W2, b2, Wih, Whh, bih, bhh):
    row = edge_index[0]
    col = edge_index[1]

    We_big = jnp.kron(jnp.eye(8, dtype=We.dtype), We)
    ef2 = edge_feats.reshape(E // 8, 8 * DE)
    w = _ew_tc(ef2, We_big, be.reshape(1, 1)).reshape(E)

    nodes = jnp.arange(N, dtype=row.dtype)
    pad = EP - EF
    spread = jnp.arange(pad, dtype=row.dtype) % N
    row_f = jnp.concatenate([row, nodes, spread])
    col_f = jnp.concatenate([col, nodes, spread])
    w_f = jnp.concatenate([w, jnp.ones((N,), w.dtype), jnp.zeros((pad,), w.dtype)])

    deg_part = _deg_sc(col_f, w_f)
    dinv = _dinv_tc(deg_part).reshape(NP)
    norm = _norm_sc(row_f, col_f, w_f, dinv)

    xp = jnp.pad(node_feats, ((0, 0), (0, NP - N), (0, 0)))

    row2 = row_f.reshape(EP // C, C)
    col2 = col_f.reshape(EP // C, C)
    nrm2 = norm.reshape(EP // C, C)

    xw1 = _mm_tc(xp, W1)
    p1 = _spmm_sc(xw1.reshape(T * NP, H), row2, col2, nrm2)
    xw2 = _cmb_tc(p1, b1.reshape(1, H), W2)
    p2 = _spmm_sc(xw2.reshape(T * NP, H), row2, col2, nrm2)

    seq = _gru_tc(p2, xp, b2.reshape(1, H), Wih.T, Whh.T,
                  bih.reshape(1, 3 * H), bhh.reshape(1, 3 * H))
    return seq[:, :N, :]


# speedup vs baseline: 29.7325x; 1.0067x over previous
import functools

import jax
import jax.numpy as jnp
from jax import lax
from jax.experimental import pallas as pl
from jax.experimental.pallas import tpu as pltpu
from jax.experimental.pallas import tpu_sc as plsc

N = 10000
E = 320000
T = 8
D = 128
DE = 16
H = 128

NP = 10240
NC = 2
NS = 16
NW = NC * NS
C = 96
EF = E + N
EW = 11520
EP = EW * NW
CH = EW // C
RPT = NP // NS
AR = 10112
RPA = AR // NS

_SC_MESH = plsc.VectorSubcoreMesh(core_axis_name="c", subcore_axis_name="s")
_SC_PARAMS = pltpu.CompilerParams(needs_layout_passes=False)

BN = 2048
NBK = NP // BN


CB = 1152
NCB = EW // CB


def _deg_body(col_hbm, w_hbm, out_hbm, colbuf, wbuf, acc):
    cid = lax.axis_index("c")
    sid = lax.axis_index("s")
    wid = cid * NS + sid

    def _zero(i, _):
        acc[pl.ds(i * 16, 16)] = jnp.zeros((16,), jnp.float32)
        return 0

    lax.fori_loop(0, NP // 16, _zero, 0)

    def _blk(g, _):
        base = wid * EW + g * CB
        pltpu.sync_copy(col_hbm.at[pl.ds(base, CB)], colbuf)
        pltpu.sync_copy(w_hbm.at[pl.ds(base, CB)], wbuf)

        def _grp(k, _):
            idx = colbuf[pl.ds(k * 16, 16)]
            val = wbuf[pl.ds(k * 16, 16)]
            plsc.addupdate_scatter(acc, [idx], val)
            return 0

        lax.fori_loop(0, CB // 16, _grp, 0)
        return 0

    lax.fori_loop(0, NCB, _blk, 0)
    pltpu.sync_copy(acc, out_hbm.at[wid])


_deg_sc = functools.partial(
    pl.kernel,
    _deg_body,
    out_type=jax.ShapeDtypeStruct((NW, NP), jnp.float32),
    mesh=_SC_MESH,
    scratch_types=[
        pltpu.VMEM((CB,), jnp.int32),
        pltpu.VMEM((CB,), jnp.float32),
        pltpu.VMEM((NP,), jnp.float32),
    ],
    compiler_params=_SC_PARAMS,
)()


def _norm_body(row_hbm, col_hbm, w_hbm, dinv_hbm, out_hbm,
               rowbuf, colbuf, wbuf, nbuf, dinv_v):
    cid = lax.axis_index("c")
    sid = lax.axis_index("s")
    wid = cid * NS + sid
    pltpu.sync_copy(dinv_hbm, dinv_v)

    def _blk(g, _):
        base = wid * EW + g * CB
        pltpu.sync_copy(row_hbm.at[pl.ds(base, CB)], rowbuf)
        pltpu.sync_copy(col_hbm.at[pl.ds(base, CB)], colbuf)
        pltpu.sync_copy(w_hbm.at[pl.ds(base, CB)], wbuf)

        def _grp(k, _):
            sl = pl.ds(k * 16, 16)
            dr = plsc.load_gather(dinv_v, [rowbuf[sl]])
            dc = plsc.load_gather(dinv_v, [colbuf[sl]])
            nbuf[sl] = dr * wbuf[sl] * dc
            return 0

        lax.fori_loop(0, CB // 16, _grp, 0)
        pltpu.sync_copy(nbuf, out_hbm.at[pl.ds(base, CB)])
        return 0

    lax.fori_loop(0, NCB, _blk, 0)


_norm_sc = functools.partial(
    pl.kernel,
    _norm_body,
    out_type=jax.ShapeDtypeStruct((EP,), jnp.float32),
    mesh=_SC_MESH,
    scratch_types=[
        pltpu.VMEM((CB,), jnp.int32),
        pltpu.VMEM((CB,), jnp.int32),
        pltpu.VMEM((CB,), jnp.float32),
        pltpu.VMEM((CB,), jnp.float32),
        pltpu.VMEM((NP,), jnp.float32),
    ],
    compiler_params=_SC_PARAMS,
)()


BLK = 24
NBL = CH // BLK


def _spmm_body(xw_hbm, row_hbm, col_hbm, norm_hbm, out_hbm,
               rowblk, colblk, nrmblk, idx_a, idx_b, idx_c,
               gbuf_a, gbuf_b, gbuf_c, acc,
               gsem_a, gsem_b, gsem_c, ssem_a, ssem_b, ssem_c):
    cid = lax.axis_index("c")
    sid = lax.axis_index("s")
    wid = cid * NS + sid
    wrow = wid * CH

    idx = (idx_a, idx_b, idx_c)
    gbufs = (gbuf_a, gbuf_b, gbuf_c)
    gsems = (gsem_a, gsem_b, gsem_c)
    ssems = (ssem_a, ssem_b, ssem_c)

    def _load_issue(t, lc, s):
        def _off(k, _):
            sl = pl.ds(k * 16, 16)
            idx[s][sl] = rowblk[lc, sl] + t * NP
            return 0

        lax.fori_loop(0, C // 16, _off, 0)
        pltpu.async_copy(xw_hbm.at[idx[s]], gbufs[s], gsems[s])

    def _scale(lc, s):
        def _grp(k, _):
            nv = nrmblk[lc, pl.ds(k * 16, 16)]
            for l in range(16):
                nsplat = jnp.broadcast_to(nv[l], (16,))
                for j in range(H // 16):
                    sl = pl.ds(j * 16, 16)
                    gbufs[s][k * 16 + l, sl] = gbufs[s][k * 16 + l, sl] * nsplat
            return 0

        lax.fori_loop(0, C // 16, _grp, 0)

    def _wait_gather(s):
        pltpu.make_async_copy(xw_hbm.at[idx[s]], gbufs[s], gsems[s]).wait()

    def _issue_scatter(lc, s):
        pltpu.async_copy(gbufs[s], acc.at[colblk.at[lc]], ssems[s], add=True)

    def _wait_scatter(lc, s):
        pltpu.make_async_copy(gbufs[s], acc.at[colblk.at[lc]], ssems[s]).wait()

    def _step(t, _):
        def _zrow(i, _):
            for j in range(H // 16):
                gbuf_a[i, pl.ds(j * 16, 16)] = jnp.zeros((16,), jnp.float32)
            return 0

        lax.fori_loop(0, C, _zrow, 0)

        def _zcp(i, _):
            pltpu.sync_copy(gbuf_a, acc.at[pl.ds(sid * RPA + i * C, C)])
            return 0

        lax.fori_loop(0, RPA // C, _zcp, 0)
        pltpu.sync_copy(gbuf_a.at[pl.ds(0, RPA - (RPA // C) * C)],
                        acc.at[pl.ds(sid * RPA + (RPA // C) * C,
                                     RPA - (RPA // C) * C)])
        plsc.subcore_barrier()

        def _blk(b, _):
            brow = wrow + b * BLK
            pltpu.sync_copy(row_hbm.at[pl.ds(brow, BLK)], rowblk)
            pltpu.sync_copy(col_hbm.at[pl.ds(brow, BLK)], colblk)
            pltpu.sync_copy(norm_hbm.at[pl.ds(brow, BLK)], nrmblk)
            _load_issue(t, 0, 0)
            _load_issue(t, 1, 1)

            def _tri(q, _):
                for m in range(3):
                    c = 3 * q + m
                    s2 = (m + 2) % 3

                    _wait_gather(m)
                    _scale(c, m)
                    _issue_scatter(c, m)

                    if m == 0:
                        @pl.when(q >= 1)
                        def _():
                            _wait_scatter(c - 1, s2)

                        _load_issue(t, c + 2, s2)
                    else:
                        _wait_scatter(c - 1, s2)

                        @pl.when(q < BLK // 3 - 1)
                        def _():
                            _load_issue(t, c + 2, s2)
                return 0

            lax.fori_loop(0, BLK // 3, _tri, 0)
            _wait_scatter(BLK - 1, 2)
            return 0

        lax.fori_loop(0, NBL, _blk, 0)
        plsc.subcore_barrier()
        pltpu.sync_copy(acc.at[pl.ds(sid * RPA, RPA)],
                        out_hbm.at[cid, t, pl.ds(sid * RPA, RPA)])
        return 0

    lax.fori_loop(0, T, _step, 0)


_spmm_sc = functools.partial(
    pl.kernel,
    _spmm_body,
    out_type=jax.ShapeDtypeStruct((NC, T, NP, H), jnp.float32),
    mesh=_SC_MESH,
    scratch_types=[
        pltpu.VMEM((BLK, C), jnp.int32),
        pltpu.VMEM((BLK, C), jnp.int32),
        pltpu.VMEM((BLK, C), jnp.float32),
        pltpu.VMEM((C,), jnp.int32),
        pltpu.VMEM((C,), jnp.int32),
        pltpu.VMEM((C,), jnp.int32),
        pltpu.VMEM((C, H), jnp.float32),
        pltpu.VMEM((C, H), jnp.float32),
        pltpu.VMEM((C, H), jnp.float32),
        pltpu.VMEM_SHARED((AR, H), jnp.float32),
        pltpu.SemaphoreType.DMA,
        pltpu.SemaphoreType.DMA,
        pltpu.SemaphoreType.DMA,
        pltpu.SemaphoreType.DMA,
        pltpu.SemaphoreType.DMA,
        pltpu.SemaphoreType.DMA,
    ],
    compiler_params=_SC_PARAMS,
)()


def _ew_body(ef_ref, web_ref, be_ref, out_ref):
    out_ref[...] = jnp.logaddexp(
        jnp.dot(ef_ref[...], web_ref[...], preferred_element_type=jnp.float32)
        + be_ref[0, 0], 0.0)


def _ew_tc(ef2, We_big, be):
    RB = 4000
    nb = ef2.shape[0] // RB
    return pl.pallas_call(
        _ew_body,
        grid=(nb,),
        in_specs=[
            pl.BlockSpec((RB, 8 * DE), lambda i: (i, 0)),
            pl.BlockSpec((8 * DE, 8), lambda i: (0, 0)),
            pl.BlockSpec((1, 1), lambda i: (0, 0), memory_space=pltpu.SMEM),
        ],
        out_specs=pl.BlockSpec((RB, 8), lambda i: (i, 0)),
        out_shape=jax.ShapeDtypeStruct((ef2.shape[0], 8), jnp.float32),
    )(ef2, We_big, be)


def _dinv_body(dp_ref, out_ref):
    deg = jnp.sum(dp_ref[...], axis=0, keepdims=True)
    out_ref[...] = jnp.where(deg > 0, jax.lax.rsqrt(deg), 0.0)


def _dinv_tc(deg_part):
    return pl.pallas_call(
        _dinv_body,
        grid=(NBK,),
        in_specs=[pl.BlockSpec((NW, BN), lambda i: (0, i))],
        out_specs=pl.BlockSpec((1, BN), lambda i: (0, i)),
        out_shape=jax.ShapeDtypeStruct((1, NP), jnp.float32),
    )(deg_part)


def _mm_body(x_ref, w_ref, out_ref):
    out_ref[0] = jnp.dot(x_ref[0], w_ref[...], preferred_element_type=jnp.float32)


def _mm_tc(xp, W):
    return pl.pallas_call(
        _mm_body,
        grid=(T, NBK),
        in_specs=[
            pl.BlockSpec((1, BN, H), lambda t, nb: (t, nb, 0)),
            pl.BlockSpec((H, H), lambda t, nb: (0, 0)),
        ],
        out_specs=pl.BlockSpec((1, BN, H), lambda t, nb: (t, nb, 0)),
        out_shape=jax.ShapeDtypeStruct((T, NP, H), jnp.float32),
    )(xp, W)


def _cmb_body(p_ref, b_ref, w_ref, out_ref):
    h1 = jnp.tanh(p_ref[0, 0] + p_ref[1, 0] + b_ref[...])
    out_ref[0] = jnp.dot(h1, w_ref[...], preferred_element_type=jnp.float32)


def _cmb_tc(parts, b1, W2):
    return pl.pallas_call(
        _cmb_body,
        grid=(T, NBK),
        in_specs=[
            pl.BlockSpec((NC, 1, BN, H), lambda t, nb: (0, t, nb, 0)),
            pl.BlockSpec((1, H), lambda t, nb: (0, 0)),
            pl.BlockSpec((H, H), lambda t, nb: (0, 0)),
        ],
        out_specs=pl.BlockSpec((1, BN, H), lambda t, nb: (t, nb, 0)),
        out_shape=jax.ShapeDtypeStruct((T, NP, H), jnp.float32),
    )(parts, b1, W2)


def _gru_body(p_ref, x_ref, b_ref, wih_ref, whh_ref, bih_ref, bhh_ref,
              out_ref, h_ref):
    t = pl.program_id(0)
    nb = pl.program_id(1)
    g = p_ref[0, 0] + p_ref[1, 0] + b_ref[...] + x_ref[0]
    h = jnp.where(t == 0, jnp.zeros_like(h_ref[nb]), h_ref[nb])
    gi = jnp.dot(g, wih_ref[...], preferred_element_type=jnp.float32) + bih_ref[...]
    gh = jnp.dot(h, whh_ref[...], preferred_element_type=jnp.float32) + bhh_ref[...]
    i_r, i_z, i_n = gi[:, :H], gi[:, H:2 * H], gi[:, 2 * H:]
    h_r, h_z, h_n = gh[:, :H], gh[:, H:2 * H], gh[:, 2 * H:]
    r = jax.nn.sigmoid(i_r + h_r)
    z = jax.nn.sigmoid(i_z + h_z)
    n = jnp.tanh(i_n + r * h_n)
    hn = (1.0 - z) * n + z * h
    h_ref[nb] = hn
    out_ref[0] = hn


def _gru_tc(parts, xp, b2, WihT, WhhT, bih, bhh):
    return pl.pallas_call(
        _gru_body,
        grid=(T, NBK),
        in_specs=[
            pl.BlockSpec((NC, 1, BN, H), lambda t, nb: (0, t, nb, 0)),
            pl.BlockSpec((1, BN, H), lambda t, nb: (t, nb, 0)),
            pl.BlockSpec((1, H), lambda t, nb: (0, 0)),
            pl.BlockSpec((H, 3 * H), lambda t, nb: (0, 0)),
            pl.BlockSpec((H, 3 * H), lambda t, nb: (0, 0)),
            pl.BlockSpec((1, 3 * H), lambda t, nb: (0, 0)),
            pl.BlockSpec((1, 3 * H), lambda t, nb: (0, 0)),
        ],
        out_specs=pl.BlockSpec((1, BN, H), lambda t, nb: (t, nb, 0)),
        out_shape=jax.ShapeDtypeStruct((T, NP, H), jnp.float32),
        scratch_shapes=[pltpu.VMEM((NBK, BN, H), jnp.float32)],
    )(parts, xp, b2, WihT, WhhT, bih, bhh)


def kernel(edge_index, edge_feats, node_feats, We, be, W1, b1, W2, b2, Wih, Whh, bih, bhh):
    row = edge_index[0]
    col = edge_index[1]

    We_big = jnp.kron(jnp.eye(8, dtype=We.dtype), We)
    ef2 = edge_feats.reshape(E // 8, 8 * DE)
    w = _ew_tc(ef2, We_big, be.reshape(1, 1)).reshape(E)

    nodes = jnp.arange(N, dtype=row.dtype)
    pad = EP - EF
    spread = jnp.arange(pad, dtype=row.dtype) % N
    row_f = jnp.concatenate([row, nodes, spread])
    col_f = jnp.concatenate([col, nodes, spread])
    w_f = jnp.concatenate([w, jnp.ones((N,), w.dtype), jnp.zeros((pad,), w.dtype)])

    deg_part = _deg_sc(col_f, w_f)
    dinv = _dinv_tc(deg_part).reshape(NP)
    norm = _norm_sc(row_f, col_f, w_f, dinv)

    xp = jnp.pad(node_feats, ((0, 0), (0, NP - N), (0, 0)))

    row2 = row_f.reshape(EP // C, C)
    col2 = col_f.reshape(EP // C, C)
    nrm2 = norm.reshape(EP // C, C)

    xw1 = _mm_tc(xp, W1)
    p1 = _spmm_sc(xw1.reshape(T * NP, H), row2, col2, nrm2)
    xw2 = _cmb_tc(p1, b1.reshape(1, H), W2)
    p2 = _spmm_sc(xw2.reshape(T * NP, H), row2, col2, nrm2)

    seq = _gru_tc(p2, xp, b2.reshape(1, H), Wih.T, Whh.T,
                  bih.reshape(1, 3 * H), bhh.reshape(1, 3 * H))
    return seq[:, :N, :]
